# baseline (speedup 1.0000x reference)
import math
import numpy as np
import jax
import jax.numpy as jnp
from jax import lax
from jax.experimental import pallas as pl
from jax.experimental.pallas import tpu as pltpu
from jax.experimental.pallas import tpu_sc as plsc

CUTOFF = 5.0
LN2 = math.log(2.0)
NC = 2
NS = 16
NW = NC * NS
CHUNK = 128
LANE = 16


def _softplus(x):
    return jnp.maximum(x, 0.0) + jnp.log(1.0 + jnp.exp(-jnp.abs(x)))


def _interleave_perm(H):
    P = np.empty(H, np.int32)
    for m in range(H // 32):
        for t in range(16):
            P[32 * m + 2 * t] = 32 * m + t
            P[32 * m + 2 * t + 1] = 32 * m + 16 + t
    return P


def _largest_div(n, cap, mult=1):
    for d in range(min(n, cap), 0, -1):
        if n % d == 0 and d % mult == 0:
            return d
    return 1


def _d2_call(row3, col3, ox3, oy3, oz3, px, py, pz):
    NWv, GRP, _ = row3.shape
    mesh = plsc.VectorSubcoreMesh(core_axis_name="c", subcore_axis_name="s")

    def body(row_hbm, col_hbm, ox_hbm, oy_hbm, oz_hbm, px_hbm, py_hbm, pz_hbm,
             d2_hbm, px_v, py_v, pz_v, row_v, col_v, ox_v, oy_v, oz_v, d2_v):
        c = lax.axis_index("c")
        s = lax.axis_index("s")
        wid = s * NC + c
        pltpu.sync_copy(px_hbm, px_v)
        pltpu.sync_copy(py_hbm, py_v)
        pltpu.sync_copy(pz_hbm, pz_v)
        pltpu.sync_copy(row_hbm.at[wid], row_v)
        pltpu.sync_copy(col_hbm.at[wid], col_v)
        pltpu.sync_copy(ox_hbm.at[wid], ox_v)
        pltpu.sync_copy(oy_hbm.at[wid], oy_v)
        pltpu.sync_copy(oz_hbm.at[wid], oz_v)

        def step(i, _):
            r = row_v[i]
            cc = col_v[i]
            rx = plsc.load_gather(px_v, [r])
            ry = plsc.load_gather(py_v, [r])
            rz = plsc.load_gather(pz_v, [r])
            cx = plsc.load_gather(px_v, [cc])
            cy = plsc.load_gather(py_v, [cc])
            cz = plsc.load_gather(pz_v, [cc])
            dx = cx + ox_v[i] - rx
            dy = cy + oy_v[i] - ry
            dz = cz + oz_v[i] - rz
            d2_v[i] = dx * dx + dy * dy + dz * dz
            return 0

        lax.fori_loop(0, GRP, step, 0)
        pltpu.sync_copy(d2_v, d2_hbm.at[wid])

    Np = px.shape[0]
    k = pl.kernel(
        body,
        out_type=jax.ShapeDtypeStruct((NWv, GRP, LANE), jnp.float32),
        mesh=mesh,
        compiler_params=pltpu.CompilerParams(needs_layout_passes=False,
                                             use_tc_tiling_on_sc=False),
        scratch_types=[
            pltpu.VMEM((Np,), jnp.float32),
            pltpu.VMEM((Np,), jnp.float32),
            pltpu.VMEM((Np,), jnp.float32),
            pltpu.VMEM((GRP, LANE), jnp.int32),
            pltpu.VMEM((GRP, LANE), jnp.int32),
            pltpu.VMEM((GRP, LANE), jnp.float32),
            pltpu.VMEM((GRP, LANE), jnp.float32),
            pltpu.VMEM((GRP, LANE), jnp.float32),
            pltpu.VMEM((GRP, LANE), jnp.float32),
        ],
    )
    return k(row3, col3, ox3, oy3, oz3, px, py, pz)


def _filters_call(d2r, w1t, b1r, w2t, b2r, E, L, G, GP, H, E_PAD):
    BE = 4096
    nblk = E_PAD // BE

    def body(d2_ref, w1t_ref, b1_ref, w2t_ref, b2_ref, wout_ref):
        b = pl.program_id(1)
        d2 = d2_ref[...].reshape(BE)
        dist = jnp.sqrt(d2)
        step = CUTOFF / (G - 1)
        offs = lax.broadcasted_iota(jnp.int32, (BE, GP), 1).astype(jnp.float32) * step
        coeff = -0.5 / (step * step)
        emb = jnp.exp(coeff * (dist[:, None] - offs) ** 2)
        h1 = jnp.dot(emb, w1t_ref[0], preferred_element_type=jnp.float32)
        h1 = _softplus(h1 + b1_ref[0]) - LN2
        Wf = jnp.dot(h1, w2t_ref[0], preferred_element_type=jnp.float32) + b2_ref[0]
        Cc = 0.5 * (jnp.cos(dist * (math.pi / CUTOFF)) + 1.0)
        eidx = b * BE + lax.broadcasted_iota(jnp.int32, (BE,), 0)
        Cc = jnp.where(eidx < E, Cc, 0.0)
        wout_ref[0] = (Wf * Cc[:, None]).astype(jnp.bfloat16)

    grid = (L, nblk)
    return pl.pallas_call(
        body,
        grid=grid,
        in_specs=[
            pl.BlockSpec((BE // 512, 512), lambda l, b: (b, 0)),
            pl.BlockSpec((1, GP, H), lambda l, b: (l, 0, 0)),
            pl.BlockSpec((1, 1, H), lambda l, b: (l, 0, 0)),
            pl.BlockSpec((1, H, H), lambda l, b: (l, 0, 0)),
            pl.BlockSpec((1, 1, H), lambda l, b: (l, 0, 0)),
        ],
        out_specs=pl.BlockSpec((1, BE, H), lambda l, b: (l, b, 0)),
        out_shape=jax.ShapeDtypeStruct((L, E_PAD, H), jnp.bfloat16),
    )(d2r, w1t, b1r, w2t, b2r)


def _message_call(w_e, vh, rowC, colC, N, H, CPT, TPW):
    mesh = plsc.VectorSubcoreMesh(core_axis_name="c", subcore_axis_name="s")
    ROWS_PT = N // NS
    ZC = _largest_div(ROWS_PT, CHUNK)

    def body(w_hbm, vh_hbm, row_hbm, col_hbm, out_hbm, accum_sh,
             ri0, ri1, ci0, ci1, wv0, wv1, gv0, gv1, prod_v,
             sr0, sr1, sc0, sc1, sw0, sw1, sg0, sg1):
        c = lax.axis_index("c")
        s = lax.axis_index("s")
        wid = s * NC + c
        RI = (ri0, ri1)
        CI = (ci0, ci1)
        WV = (wv0, wv1)
        GV = (gv0, gv1)
        SR = (sr0, sr1)
        SC = (sc0, sc1)
        SW = (sw0, sw1)
        SG = (sg0, sg1)
        zz = jnp.zeros((LANE,), jnp.float32)

        def zrow(i, _):
            for j in range(H // LANE):
                prod_v[i, pl.ds(j * LANE, LANE)] = zz
            return 0

        lax.fori_loop(0, CHUNK, zrow, 0)
        rbase = s * ROWS_PT
        for t in range(ROWS_PT // ZC):
            pltpu.sync_copy(prod_v.at[pl.ds(0, ZC)],
                            accum_sh.at[pl.ds(rbase + t * ZC, ZC)])
        plsc.subcore_barrier()
        ebase = wid * TPW

        def start_idx_w(k, b):
            pltpu.async_copy(row_hbm.at[wid].at[pl.ds(k, 1)], RI[b], SR[b])
            pltpu.async_copy(col_hbm.at[wid].at[pl.ds(k, 1)], CI[b], SC[b])
            pltpu.async_copy(w_hbm.at[pl.ds(ebase + k * CHUNK, CHUNK)], WV[b], SW[b])

        def wait_r_start_gather(b):
            pltpu.make_async_copy(row_hbm.at[wid].at[pl.ds(0, 1)], RI[b], SR[b]).wait()
            pltpu.async_copy(vh_hbm.at[RI[b].at[0]], GV[b], SG[b])

        def process(b):
            pltpu.make_async_copy(w_hbm.at[pl.ds(0, CHUNK)], WV[b], SW[b]).wait()
            pltpu.make_async_copy(vh_hbm.at[pl.ds(0, CHUNK)], GV[b], SG[b]).wait()

            def mrow(i, _):
                for m in range(H // 32):
                    wb = WV[b][i, pl.ds(32 * m, 32)]
                    vb = GV[b][i, pl.ds(32 * m, 32)]
                    w0, w1 = plsc.unpack(wb, format=plsc.PackFormat.INTERLEAVED)
                    v0, v1 = plsc.unpack(vb, format=plsc.PackFormat.INTERLEAVED)
                    prod_v[i, pl.ds(32 * m, LANE)] = w0 * v0
                    prod_v[i, pl.ds(32 * m + LANE, LANE)] = w1 * v1
                return 0

            lax.fori_loop(0, CHUNK, mrow, 0)
            pltpu.make_async_copy(col_hbm.at[wid].at[pl.ds(0, 1)], CI[b], SC[b]).wait()
            pltpu.sync_copy(prod_v, accum_sh.at[CI[b].at[0]], add=True)

        start_idx_w(0, 0)
        start_idx_w(1, 1)
        wait_r_start_gather(0)

        def body2(j, _):
            k = 2 * j
            wait_r_start_gather(1)
            process(0)

            @pl.when(k + 2 < CPT)
            def _():
                start_idx_w(k + 2, 0)
            process(1)

            @pl.when(k + 3 < CPT)
            def _():
                start_idx_w(k + 3, 1)

            @pl.when(k + 2 < CPT)
            def _():
                wait_r_start_gather(0)
            return 0

        lax.fori_loop(0, CPT // 2, body2, 0)
        plsc.subcore_barrier()
        pltpu.sync_copy(accum_sh.at[pl.ds(rbase, ROWS_PT)],
                        out_hbm.at[c, pl.ds(rbase, ROWS_PT)])

    k = pl.kernel(
        body,
        out_type=jax.ShapeDtypeStruct((NC, N, H), jnp.float32),
        mesh=mesh,
        compiler_params=pltpu.CompilerParams(needs_layout_passes=False,
                                             use_tc_tiling_on_sc=False),
        scratch_types=[
            pltpu.VMEM_SHARED((N, H), jnp.float32),
            pltpu.VMEM((1, CHUNK), jnp.int32),
            pltpu.VMEM((1, CHUNK), jnp.int32),
            pltpu.VMEM((1, CHUNK), jnp.int32),
            pltpu.VMEM((1, CHUNK), jnp.int32),
            pltpu.VMEM((CHUNK, H), jnp.bfloat16),
            pltpu.VMEM((CHUNK, H), jnp.bfloat16),
            pltpu.VMEM((CHUNK, H), jnp.bfloat16),
            pltpu.VMEM((CHUNK, H), jnp.bfloat16),
            pltpu.VMEM((CHUNK, H), jnp.float32),
        ] + [pltpu.SemaphoreType.DMA] * 8,
    )
    return k(w_e, vh, rowC, colC)


def _update_call(part, v, w1t, b1, w2t, b2, lint, N, H):
    BN = _largest_div(N, 1024, mult=8)

    def body(p_ref, v_ref, w1_ref, b1_ref, w2_ref, b2_ref, lt_ref, vn_ref, vh_ref):
        out = p_ref[0] + p_ref[1]
        h = _softplus(jnp.dot(out, w1_ref[...], preferred_element_type=jnp.float32)
                      + b1_ref[...]) - LN2
        upd = jnp.dot(h, w2_ref[...], preferred_element_type=jnp.float32) + b2_ref[...]
        vn = v_ref[...] + upd
        vn_ref[...] = vn
        vh_ref[...] = jnp.dot(vn, lt_ref[...],
                              preferred_element_type=jnp.float32).astype(jnp.bfloat16)

    grid = (N // BN,)
    return pl.pallas_call(
        body,
        grid=grid,
        in_specs=[
            pl.BlockSpec((2, BN, H), lambda b: (0, b, 0)),
            pl.BlockSpec((BN, H), lambda b: (b, 0)),
            pl.BlockSpec((H, H), lambda b: (0, 0)),
            pl.BlockSpec((1, H), lambda b: (0, 0)),
            pl.BlockSpec((H, H), lambda b: (0, 0)),
            pl.BlockSpec((1, H), lambda b: (0, 0)),
            pl.BlockSpec((H, H), lambda b: (0, 0)),
        ],
        out_specs=[
            pl.BlockSpec((BN, H), lambda b: (b, 0)),
            pl.BlockSpec((BN, H), lambda b: (b, 0)),
        ],
        out_shape=[
            jax.ShapeDtypeStruct((N, H), jnp.float32),
            jax.ShapeDtypeStruct((N, H), jnp.bfloat16),
        ],
    )(part, v, w1t, b1, w2t, b2, lint)


def _vh0_call(v, lint, N, H):
    BN = _largest_div(N, 1024, mult=8)

    def body(v_ref, lt_ref, vh_ref):
        vh_ref[...] = jnp.dot(v_ref[...], lt_ref[...],
                              preferred_element_type=jnp.float32).astype(jnp.bfloat16)

    return pl.pallas_call(
        body,
        grid=(N // BN,),
        in_specs=[
            pl.BlockSpec((BN, H), lambda b: (b, 0)),
            pl.BlockSpec((H, H), lambda b: (0, 0)),
        ],
        out_specs=pl.BlockSpec((BN, H), lambda b: (b, 0)),
        out_shape=jax.ShapeDtypeStruct((N, H), jnp.bfloat16),
    )(v, lint)


def kernel(v, pos, edges, offsets_real, lin_w, mlp_w1, mlp_b1, mlp_w2, mlp_b2,
           v_w1, v_b1, v_w2, v_b2):
    N, H = v.shape
    L, FLT, G = mlp_w1.shape
    E = edges.shape[1]
    assert N % NS == 0 and H % LANE == 0

    CPT = -(-E // (NW * CHUNK))
    CPT += CPT % 2
    TPW = CPT * CHUNK
    E_PAD = NW * TPW
    GRP = TPW // LANE
    GP = -(-G // 8) * 8

    f32 = jnp.float32
    row = jnp.pad(edges[0], (0, E_PAD - E)).astype(jnp.int32)
    col = jnp.pad(edges[1], (0, E_PAD - E)).astype(jnp.int32)
    offp = jnp.pad(offsets_real, ((0, E_PAD - E), (0, 0))).astype(f32)
    row3 = row.reshape(NW, GRP, LANE)
    col3 = col.reshape(NW, GRP, LANE)
    ox3 = offp[:, 0].reshape(NW, GRP, LANE)
    oy3 = offp[:, 1].reshape(NW, GRP, LANE)
    oz3 = offp[:, 2].reshape(NW, GRP, LANE)
    px = pos[:, 0].astype(f32)
    py = pos[:, 1].astype(f32)
    pz = pos[:, 2].astype(f32)

    d2 = _d2_call(row3, col3, ox3, oy3, oz3, px, py, pz)
    d2r = d2.reshape(E_PAD // 512, 512)

    P2 = _interleave_perm(H)
    w1t = jnp.pad(jnp.swapaxes(mlp_w1, 1, 2), ((0, 0), (0, GP - G), (0, 0))).astype(f32)
    b1r = mlp_b1.reshape(L, 1, FLT).astype(f32)
    w2t = jnp.swapaxes(mlp_w2, 1, 2).astype(f32)[:, :, P2]
    b2r = mlp_b2.reshape(L, 1, FLT).astype(f32)[:, :, P2]
    W_all = _filters_call(d2r, w1t, b1r, w2t, b2r, E, L, G, GP, H, E_PAD)

    rowC = row.reshape(NW, CPT, CHUNK)
    colC = col.reshape(NW, CPT, CHUNK)

    v = v.astype(f32)
    vh = _vh0_call(v, jnp.swapaxes(lin_w[0], 0, 1).astype(f32)[:, P2], N, H)
    for l in range(L):
        part = _message_call(W_all[l], vh, rowC, colC, N, H, CPT, TPW)
        lint = jnp.swapaxes(lin_w[(l + 1) % L], 0, 1).astype(f32)[:, P2]
        v, vh = _update_call(part, v,
                             jnp.swapaxes(v_w1[l], 0, 1).astype(f32),
                             v_b1[l].reshape(1, H).astype(f32),
                             jnp.swapaxes(v_w2[l], 0, 1).astype(f32),
                             v_b2[l].reshape(1, H).astype(f32),
                             lint, N, H)
    return v


# speedup vs baseline: 1.0983x; 1.0983x over previous
import math
import numpy as np
import jax
import jax.numpy as jnp
from jax import lax
from jax.experimental import pallas as pl
from jax.experimental.pallas import tpu as pltpu
from jax.experimental.pallas import tpu_sc as plsc

CUTOFF = 5.0
LN2 = math.log(2.0)
NC = 2
NS = 16
NW = NC * NS
CHUNK = 128
LANE = 16


def _softplus(x):
    return jnp.maximum(x, 0.0) + jnp.log(1.0 + jnp.exp(-jnp.abs(x)))


def _pack_pair(a, b):
    pa = lax.bitcast_convert_type(a.astype(jnp.bfloat16), jnp.uint16).astype(jnp.uint32)
    pb = lax.bitcast_convert_type(b.astype(jnp.bfloat16), jnp.uint16).astype(jnp.uint32)
    return lax.bitcast_convert_type(pa | (pb << 16), jnp.int32)


def _interleave_perm(H):
    P = np.empty(H, np.int32)
    for m in range(H // 32):
        for t in range(16):
            P[32 * m + 2 * t] = 32 * m + t
            P[32 * m + 2 * t + 1] = 32 * m + 16 + t
    return P


def _largest_div(n, cap, mult=1):
    for d in range(min(n, cap), 0, -1):
        if n % d == 0 and d % mult == 0:
            return d
    return 1


def _d2_call(row3, col3, ox3, oy3, oz3, px, py, pz):
    NWv, GRP, _ = row3.shape
    mesh = plsc.VectorSubcoreMesh(core_axis_name="c", subcore_axis_name="s")

    def body(row_hbm, col_hbm, ox_hbm, oy_hbm, oz_hbm, px_hbm, py_hbm, pz_hbm,
             d2_hbm, px_v, py_v, pz_v, row_v, col_v, ox_v, oy_v, oz_v, d2_v):
        c = lax.axis_index("c")
        s = lax.axis_index("s")
        wid = s * NC + c
        pltpu.sync_copy(px_hbm, px_v)
        pltpu.sync_copy(py_hbm, py_v)
        pltpu.sync_copy(pz_hbm, pz_v)
        pltpu.sync_copy(row_hbm.at[wid], row_v)
        pltpu.sync_copy(col_hbm.at[wid], col_v)
        pltpu.sync_copy(ox_hbm.at[wid], ox_v)
        pltpu.sync_copy(oy_hbm.at[wid], oy_v)
        pltpu.sync_copy(oz_hbm.at[wid], oz_v)

        def step(i, _):
            r = row_v[i]
            cc = col_v[i]
            rx = plsc.load_gather(px_v, [r])
            ry = plsc.load_gather(py_v, [r])
            rz = plsc.load_gather(pz_v, [r])
            cx = plsc.load_gather(px_v, [cc])
            cy = plsc.load_gather(py_v, [cc])
            cz = plsc.load_gather(pz_v, [cc])
            dx = cx + ox_v[i] - rx
            dy = cy + oy_v[i] - ry
            dz = cz + oz_v[i] - rz
            d2_v[i] = dx * dx + dy * dy + dz * dz
            return 0

        lax.fori_loop(0, GRP, step, 0)
        pltpu.sync_copy(d2_v, d2_hbm.at[wid])

    Np = px.shape[0]
    k = pl.kernel(
        body,
        out_type=jax.ShapeDtypeStruct((NWv, GRP, LANE), jnp.float32),
        mesh=mesh,
        compiler_params=pltpu.CompilerParams(needs_layout_passes=False,
                                             use_tc_tiling_on_sc=False),
        scratch_types=[
            pltpu.VMEM((Np,), jnp.float32),
            pltpu.VMEM((Np,), jnp.float32),
            pltpu.VMEM((Np,), jnp.float32),
            pltpu.VMEM((GRP, LANE), jnp.int32),
            pltpu.VMEM((GRP, LANE), jnp.int32),
            pltpu.VMEM((GRP, LANE), jnp.float32),
            pltpu.VMEM((GRP, LANE), jnp.float32),
            pltpu.VMEM((GRP, LANE), jnp.float32),
            pltpu.VMEM((GRP, LANE), jnp.float32),
        ],
    )
    return k(row3, col3, ox3, oy3, oz3, px, py, pz)


def _filters_call(d2r, w1t, b1r, w2t, b2r, E, L, G, GP, H, E_PAD):
    BE = 4096
    nblk = E_PAD // BE

    def body(d2_ref, w1t_ref, b1_ref, w2t_ref, b2_ref, wout_ref):
        b = pl.program_id(1)
        d2 = d2_ref[...].reshape(BE)
        dist = jnp.sqrt(d2)
        step = CUTOFF / (G - 1)
        offs = lax.broadcasted_iota(jnp.int32, (BE, GP), 1).astype(jnp.float32) * step
        coeff = -0.5 / (step * step)
        emb = jnp.exp(coeff * (dist[:, None] - offs) ** 2)
        h1 = jnp.dot(emb, w1t_ref[0], preferred_element_type=jnp.float32)
        h1 = _softplus(h1 + b1_ref[0]) - LN2
        Wf = jnp.dot(h1, w2t_ref[0], preferred_element_type=jnp.float32) + b2_ref[0]
        Cc = 0.5 * (jnp.cos(dist * (math.pi / CUTOFF)) + 1.0)
        eidx = b * BE + lax.broadcasted_iota(jnp.int32, (BE,), 0)
        Cc = jnp.where(eidx < E, Cc, 0.0)
        Wl = (Wf * Cc[:, None]).reshape(BE // 2, 2, H)
        wout_ref[0] = _pack_pair(Wl[:, 0, :], Wl[:, 1, :])

    grid = (L, nblk)
    return pl.pallas_call(
        body,
        grid=grid,
        in_specs=[
            pl.BlockSpec((BE // 512, 512), lambda l, b: (b, 0)),
            pl.BlockSpec((1, GP, H), lambda l, b: (l, 0, 0)),
            pl.BlockSpec((1, 1, H), lambda l, b: (l, 0, 0)),
            pl.BlockSpec((1, H, H), lambda l, b: (l, 0, 0)),
            pl.BlockSpec((1, 1, H), lambda l, b: (l, 0, 0)),
        ],
        out_specs=pl.BlockSpec((1, BE // 2, H), lambda l, b: (l, b, 0)),
        out_shape=jax.ShapeDtypeStruct((L, E_PAD // 2, H), jnp.int32),
    )(d2r, w1t, b1r, w2t, b2r)


def _message_call(w_e, vh, rowC, colC, N, H, CPT, TPW):
    mesh = plsc.VectorSubcoreMesh(core_axis_name="c", subcore_axis_name="s")
    ROWS_PT = N // NS
    ZC = _largest_div(ROWS_PT, CHUNK)

    def body(w_hbm, vh_hbm, row_hbm, col_hbm, out_hbm, accum_sh,
             ri0, ri1, ci0, ci1, wv0, wv1, gv0, gv1, prod_v, cs_v,
             sr0, sr1, sc0, sc1, sw0, sw1, sg0, sg1, sem_s):
        c = lax.axis_index("c")
        s = lax.axis_index("s")
        wid = s * NC + c
        RI = (ri0, ri1)
        CI = (ci0, ci1)
        WV = (wv0, wv1)
        GV = (gv0, gv1)
        SR = (sr0, sr1)
        SC = (sc0, sc1)
        SW = (sw0, sw1)
        SG = (sg0, sg1)
        zz = jnp.zeros((LANE,), jnp.float32)

        def zrow(i, _):
            for j in range(H // LANE):
                prod_v[i, pl.ds(j * LANE, LANE)] = zz
            return 0

        lax.fori_loop(0, CHUNK, zrow, 0)
        rbase = s * ROWS_PT
        for t in range(ROWS_PT // ZC):
            pltpu.sync_copy(prod_v.at[pl.ds(0, ZC)],
                            accum_sh.at[pl.ds(rbase + t * ZC, ZC)])
        plsc.subcore_barrier()
        ebase = wid * TPW

        ebase2 = wid * (TPW // 2)

        def start_idx_w(k, b):
            pltpu.async_copy(row_hbm.at[wid].at[pl.ds(k, 1)], RI[b], SR[b])
            pltpu.async_copy(col_hbm.at[wid].at[pl.ds(k, 1)], CI[b], SC[b])
            pltpu.async_copy(w_hbm.at[pl.ds(ebase2 + k * (CHUNK // 2), CHUNK // 2)],
                             WV[b], SW[b])

        def wait_r_start_gather(b):
            pltpu.make_async_copy(row_hbm.at[wid].at[pl.ds(0, 1)], RI[b], SR[b]).wait()
            pltpu.async_copy(vh_hbm.at[RI[b].at[0]], GV[b], SG[b])

        def scatter_wait():
            pltpu.make_async_copy(prod_v, accum_sh.at[cs_v.at[0]], sem_s).wait()

        def process(k, b):
            pltpu.make_async_copy(w_hbm.at[pl.ds(0, CHUNK // 2)], WV[b], SW[b]).wait()
            pltpu.make_async_copy(vh_hbm.at[pl.ds(0, CHUNK)], GV[b], SG[b]).wait()

            @pl.when(k > 0)
            def _():
                scatter_wait()

            def mrow(r, _):
                for q in range(H // 32):
                    vA = plsc.bitcast(GV[b][2 * r, pl.ds(LANE * q, LANE)], jnp.bfloat16)
                    vB = plsc.bitcast(GV[b][2 * r + 1, pl.ds(LANE * q, LANE)], jnp.bfloat16)
                    v0e, v1e = plsc.unpack(vA, format=plsc.PackFormat.INTERLEAVED)
                    v0o, v1o = plsc.unpack(vB, format=plsc.PackFormat.INTERLEAVED)
                    wA = plsc.bitcast(WV[b][r, pl.ds(32 * q, LANE)], jnp.bfloat16)
                    wB = plsc.bitcast(WV[b][r, pl.ds(32 * q + LANE, LANE)], jnp.bfloat16)
                    wAe, wAo = plsc.unpack(wA, format=plsc.PackFormat.INTERLEAVED)
                    wBe, wBo = plsc.unpack(wB, format=plsc.PackFormat.INTERLEAVED)
                    prod_v[2 * r, pl.ds(32 * q, LANE)] = wAe * v0e
                    prod_v[2 * r, pl.ds(32 * q + LANE, LANE)] = wBe * v1e
                    prod_v[2 * r + 1, pl.ds(32 * q, LANE)] = wAo * v0o
                    prod_v[2 * r + 1, pl.ds(32 * q + LANE, LANE)] = wBo * v1o
                return 0

            lax.fori_loop(0, CHUNK // 2, mrow, 0)
            pltpu.make_async_copy(col_hbm.at[wid].at[pl.ds(0, 1)], CI[b], SC[b]).wait()
            for t in range(CHUNK // LANE):
                cs_v[0, pl.ds(LANE * t, LANE)] = CI[b][0, pl.ds(LANE * t, LANE)]
            pltpu.async_copy(prod_v, accum_sh.at[cs_v.at[0]], sem_s, add=True)

        start_idx_w(0, 0)
        start_idx_w(1, 1)
        wait_r_start_gather(0)

        def body2(j, _):
            k = 2 * j
            wait_r_start_gather(1)
            process(k, 0)

            @pl.when(k + 2 < CPT)
            def _():
                start_idx_w(k + 2, 0)
            process(k + 1, 1)

            @pl.when(k + 3 < CPT)
            def _():
                start_idx_w(k + 3, 1)

            @pl.when(k + 2 < CPT)
            def _():
                wait_r_start_gather(0)
            return 0

        lax.fori_loop(0, CPT // 2, body2, 0)
        scatter_wait()
        plsc.subcore_barrier()
        pltpu.sync_copy(accum_sh.at[pl.ds(rbase, ROWS_PT)],
                        out_hbm.at[c, pl.ds(rbase, ROWS_PT)])

    k = pl.kernel(
        body,
        out_type=jax.ShapeDtypeStruct((NC, N, H), jnp.float32),
        mesh=mesh,
        compiler_params=pltpu.CompilerParams(needs_layout_passes=False,
                                             use_tc_tiling_on_sc=False),
        scratch_types=[
            pltpu.VMEM_SHARED((N, H), jnp.float32),
            pltpu.VMEM((1, CHUNK), jnp.int32),
            pltpu.VMEM((1, CHUNK), jnp.int32),
            pltpu.VMEM((1, CHUNK), jnp.int32),
            pltpu.VMEM((1, CHUNK), jnp.int32),
            pltpu.VMEM((CHUNK // 2, H), jnp.int32),
            pltpu.VMEM((CHUNK // 2, H), jnp.int32),
            pltpu.VMEM((CHUNK, H // 2), jnp.int32),
            pltpu.VMEM((CHUNK, H // 2), jnp.int32),
            pltpu.VMEM((CHUNK, H), jnp.float32),
            pltpu.VMEM((1, CHUNK), jnp.int32),
        ] + [pltpu.SemaphoreType.DMA] * 9,
    )
    return k(w_e, vh, rowC, colC)


def _update_call(part, v, w1t, b1, w2t, b2, lo, hi, N, H):
    BN = _largest_div(N, 1024, mult=8)

    def body(p_ref, v_ref, w1_ref, b1_ref, w2_ref, b2_ref, lo_ref, hi_ref,
             vn_ref, vh_ref):
        out = p_ref[0] + p_ref[1]
        h = _softplus(jnp.dot(out, w1_ref[...], preferred_element_type=jnp.float32)
                      + b1_ref[...]) - LN2
        upd = jnp.dot(h, w2_ref[...], preferred_element_type=jnp.float32) + b2_ref[...]
        vn = v_ref[...] + upd
        vn_ref[...] = vn
        vh_ref[...] = _pack_pair(
            jnp.dot(vn, lo_ref[...], preferred_element_type=jnp.float32),
            jnp.dot(vn, hi_ref[...], preferred_element_type=jnp.float32))

    grid = (N // BN,)
    return pl.pallas_call(
        body,
        grid=grid,
        in_specs=[
            pl.BlockSpec((2, BN, H), lambda b: (0, b, 0)),
            pl.BlockSpec((BN, H), lambda b: (b, 0)),
            pl.BlockSpec((H, H), lambda b: (0, 0)),
            pl.BlockSpec((1, H), lambda b: (0, 0)),
            pl.BlockSpec((H, H), lambda b: (0, 0)),
            pl.BlockSpec((1, H), lambda b: (0, 0)),
            pl.BlockSpec((H, H // 2), lambda b: (0, 0)),
            pl.BlockSpec((H, H // 2), lambda b: (0, 0)),
        ],
        out_specs=[
            pl.BlockSpec((BN, H), lambda b: (b, 0)),
            pl.BlockSpec((BN, H // 2), lambda b: (b, 0)),
        ],
        out_shape=[
            jax.ShapeDtypeStruct((N, H), jnp.float32),
            jax.ShapeDtypeStruct((N, H // 2), jnp.int32),
        ],
    )(part, v, w1t, b1, w2t, b2, lo, hi)


def _vh0_call(v, lo, hi, N, H):
    BN = _largest_div(N, 1024, mult=8)

    def body(v_ref, lo_ref, hi_ref, vh_ref):
        vh_ref[...] = _pack_pair(
            jnp.dot(v_ref[...], lo_ref[...], preferred_element_type=jnp.float32),
            jnp.dot(v_ref[...], hi_ref[...], preferred_element_type=jnp.float32))

    return pl.pallas_call(
        body,
        grid=(N // BN,),
        in_specs=[
            pl.BlockSpec((BN, H), lambda b: (b, 0)),
            pl.BlockSpec((H, H // 2), lambda b: (0, 0)),
            pl.BlockSpec((H, H // 2), lambda b: (0, 0)),
        ],
        out_specs=pl.BlockSpec((BN, H // 2), lambda b: (b, 0)),
        out_shape=jax.ShapeDtypeStruct((N, H // 2), jnp.int32),
    )(v, lo, hi)


def kernel(v, pos, edges, offsets_real, lin_w, mlp_w1, mlp_b1, mlp_w2, mlp_b2,
           v_w1, v_b1, v_w2, v_b2):
    N, H = v.shape
    L, FLT, G = mlp_w1.shape
    E = edges.shape[1]
    assert N % NS == 0 and H % LANE == 0

    CPT = -(-E // (NW * CHUNK))
    CPT += CPT % 2
    TPW = CPT * CHUNK
    E_PAD = NW * TPW
    GRP = TPW // LANE
    GP = -(-G // 8) * 8

    f32 = jnp.float32
    row = jnp.pad(edges[0], (0, E_PAD - E)).astype(jnp.int32)
    col = jnp.pad(edges[1], (0, E_PAD - E)).astype(jnp.int32)
    offp = jnp.pad(offsets_real, ((0, E_PAD - E), (0, 0))).astype(f32)
    row3 = row.reshape(NW, GRP, LANE)
    col3 = col.reshape(NW, GRP, LANE)
    ox3 = offp[:, 0].reshape(NW, GRP, LANE)
    oy3 = offp[:, 1].reshape(NW, GRP, LANE)
    oz3 = offp[:, 2].reshape(NW, GRP, LANE)
    px = pos[:, 0].astype(f32)
    py = pos[:, 1].astype(f32)
    pz = pos[:, 2].astype(f32)

    d2 = _d2_call(row3, col3, ox3, oy3, oz3, px, py, pz)
    d2r = d2.reshape(E_PAD // 512, 512)

    P2 = _interleave_perm(H)
    w1t = jnp.pad(jnp.swapaxes(mlp_w1, 1, 2), ((0, 0), (0, GP - G), (0, 0))).astype(f32)
    b1r = mlp_b1.reshape(L, 1, FLT).astype(f32)
    w2t = jnp.swapaxes(mlp_w2, 1, 2).astype(f32)
    b2r = mlp_b2.reshape(L, 1, FLT).astype(f32)
    W_all = _filters_call(d2r, w1t, b1r, w2t, b2r, E, L, G, GP, H, E_PAD)

    rowC = row.reshape(NW, CPT, CHUNK)
    colC = col.reshape(NW, CPT, CHUNK)

    v = v.astype(f32)

    def lohi(w):
        lp = jnp.swapaxes(w, 0, 1).astype(f32)[:, P2]
        return lp[:, 0::2], lp[:, 1::2]

    lo0, hi0 = lohi(lin_w[0])
    vh = _vh0_call(v, lo0, hi0, N, H)
    for l in range(L):
        part = _message_call(W_all[l], vh, rowC, colC, N, H, CPT, TPW)
        lo, hi = lohi(lin_w[(l + 1) % L])
        v, vh = _update_call(part, v,
                             jnp.swapaxes(v_w1[l], 0, 1).astype(f32),
                             v_b1[l].reshape(1, H).astype(f32),
                             jnp.swapaxes(v_w2[l], 0, 1).astype(f32),
                             v_b2[l].reshape(1, H).astype(f32),
                             lo, hi, N, H)
    return v


# speedup vs baseline: 1.3522x; 1.2312x over previous
import math
import numpy as np
import jax
import jax.numpy as jnp
from jax import lax
from jax.experimental import pallas as pl
from jax.experimental.pallas import tpu as pltpu
from jax.experimental.pallas import tpu_sc as plsc

CUTOFF = 5.0
LN2 = math.log(2.0)
NC = 2
NS = 16
NW = NC * NS
CHUNK = 128
LANE = 16


def _softplus(x):
    return jnp.maximum(x, 0.0) + jnp.log(1.0 + jnp.exp(-jnp.abs(x)))


def _pack_pair(a, b):
    pa = lax.bitcast_convert_type(a.astype(jnp.bfloat16), jnp.uint16).astype(jnp.uint32)
    pb = lax.bitcast_convert_type(b.astype(jnp.bfloat16), jnp.uint16).astype(jnp.uint32)
    return lax.bitcast_convert_type(pa | (pb << 16), jnp.int32)


def _interleave_perm(H):
    P = np.empty(H, np.int32)
    for m in range(H // 32):
        for t in range(16):
            P[32 * m + 2 * t] = 32 * m + t
            P[32 * m + 2 * t + 1] = 32 * m + 16 + t
    return P


def _largest_div(n, cap, mult=1):
    for d in range(min(n, cap), 0, -1):
        if n % d == 0 and d % mult == 0:
            return d
    return 1


def _d2_call(row3, col3, ox3, oy3, oz3, px, py, pz):
    NWv, GRP, _ = row3.shape
    mesh = plsc.VectorSubcoreMesh(core_axis_name="c", subcore_axis_name="s")

    def body(row_hbm, col_hbm, ox_hbm, oy_hbm, oz_hbm, px_hbm, py_hbm, pz_hbm,
             d2_hbm, px_v, py_v, pz_v, row_v, col_v, ox_v, oy_v, oz_v, d2_v):
        c = lax.axis_index("c")
        s = lax.axis_index("s")
        wid = s * NC + c
        pltpu.sync_copy(px_hbm, px_v)
        pltpu.sync_copy(py_hbm, py_v)
        pltpu.sync_copy(pz_hbm, pz_v)
        pltpu.sync_copy(row_hbm.at[wid], row_v)
        pltpu.sync_copy(col_hbm.at[wid], col_v)
        pltpu.sync_copy(ox_hbm.at[wid], ox_v)
        pltpu.sync_copy(oy_hbm.at[wid], oy_v)
        pltpu.sync_copy(oz_hbm.at[wid], oz_v)

        def step(i, _):
            r = row_v[i]
            cc = col_v[i]
            rx = plsc.load_gather(px_v, [r])
            ry = plsc.load_gather(py_v, [r])
            rz = plsc.load_gather(pz_v, [r])
            cx = plsc.load_gather(px_v, [cc])
            cy = plsc.load_gather(py_v, [cc])
            cz = plsc.load_gather(pz_v, [cc])
            dx = cx + ox_v[i] - rx
            dy = cy + oy_v[i] - ry
            dz = cz + oz_v[i] - rz
            d2_v[i] = dx * dx + dy * dy + dz * dz
            return 0

        lax.fori_loop(0, GRP, step, 0)
        pltpu.sync_copy(d2_v, d2_hbm.at[wid])

    Np = px.shape[0]
    k = pl.kernel(
        body,
        out_type=jax.ShapeDtypeStruct((NWv, GRP, LANE), jnp.float32),
        mesh=mesh,
        compiler_params=pltpu.CompilerParams(needs_layout_passes=False,
                                             use_tc_tiling_on_sc=False),
        scratch_types=[
            pltpu.VMEM((Np,), jnp.float32),
            pltpu.VMEM((Np,), jnp.float32),
            pltpu.VMEM((Np,), jnp.float32),
            pltpu.VMEM((GRP, LANE), jnp.int32),
            pltpu.VMEM((GRP, LANE), jnp.int32),
            pltpu.VMEM((GRP, LANE), jnp.float32),
            pltpu.VMEM((GRP, LANE), jnp.float32),
            pltpu.VMEM((GRP, LANE), jnp.float32),
            pltpu.VMEM((GRP, LANE), jnp.float32),
        ],
    )
    return k(row3, col3, ox3, oy3, oz3, px, py, pz)


def _filters_call(d2e, d2o, w1t, b1r, w2t, b2r, E, L, G, GP, H, E_PAD):
    BE = 2048
    nblk = E_PAD // 2 // BE

    def half(d2, w1, b1, w2, b2, base, step, coeff):
        dist = jnp.sqrt(d2)
        offs = lax.broadcasted_iota(jnp.int32, (BE, GP), 1).astype(jnp.float32) * step
        emb = jnp.exp(coeff * (dist[:, None] - offs) ** 2)
        h1 = jnp.dot(emb, w1, preferred_element_type=jnp.float32)
        h1 = _softplus(h1 + b1) - LN2
        Wf = jnp.dot(h1, w2, preferred_element_type=jnp.float32) + b2
        Cc = 0.5 * (jnp.cos(dist * (math.pi / CUTOFF)) + 1.0)
        eidx = base + 2 * lax.broadcasted_iota(jnp.int32, (BE,), 0)
        Cc = jnp.where(eidx < E, Cc, 0.0)
        return Wf * Cc[:, None]

    def body(d2e_ref, d2o_ref, w1t_ref, b1_ref, w2t_ref, b2_ref, wout_ref):
        b = pl.program_id(1)
        step = CUTOFF / (G - 1)
        coeff = -0.5 / (step * step)
        w1 = w1t_ref[0]
        b1 = b1_ref[0]
        w2 = w2t_ref[0]
        b2 = b2_ref[0]
        We = half(d2e_ref[...].reshape(BE), w1, b1, w2, b2, b * 2 * BE, step, coeff)
        Wo = half(d2o_ref[...].reshape(BE), w1, b1, w2, b2, b * 2 * BE + 1, step, coeff)
        wout_ref[0] = _pack_pair(We, Wo)

    grid = (L, nblk)
    return pl.pallas_call(
        body,
        grid=grid,
        in_specs=[
            pl.BlockSpec((BE // 256, 256), lambda l, b: (b, 0)),
            pl.BlockSpec((BE // 256, 256), lambda l, b: (b, 0)),
            pl.BlockSpec((1, GP, H), lambda l, b: (l, 0, 0)),
            pl.BlockSpec((1, 1, H), lambda l, b: (l, 0, 0)),
            pl.BlockSpec((1, H, H), lambda l, b: (l, 0, 0)),
            pl.BlockSpec((1, 1, H), lambda l, b: (l, 0, 0)),
        ],
        out_specs=pl.BlockSpec((1, BE, H), lambda l, b: (l, b, 0)),
        out_shape=jax.ShapeDtypeStruct((L, E_PAD // 2, H), jnp.int32),
    )(d2e, d2o, w1t, b1r, w2t, b2r)


def _message_call(w_e, vh, rowC, colC, N, H, CPT, TPW):
    mesh = plsc.VectorSubcoreMesh(core_axis_name="c", subcore_axis_name="s")
    ROWS_PT = N // NS
    ZC = _largest_div(ROWS_PT, CHUNK)

    def body(w_hbm, vh_hbm, row_hbm, col_hbm, out_hbm, accum_sh,
             ri0, ri1, ci0, ci1, wv0, wv1, gv0, gv1, prod_v, cs_v,
             sr0, sr1, sc0, sc1, sw0, sw1, sg0, sg1, sem_s):
        c = lax.axis_index("c")
        s = lax.axis_index("s")
        wid = s * NC + c
        RI = (ri0, ri1)
        CI = (ci0, ci1)
        WV = (wv0, wv1)
        GV = (gv0, gv1)
        SR = (sr0, sr1)
        SC = (sc0, sc1)
        SW = (sw0, sw1)
        SG = (sg0, sg1)
        zz = jnp.zeros((LANE,), jnp.float32)

        def zrow(i, _):
            for j in range(H // LANE):
                prod_v[i, pl.ds(j * LANE, LANE)] = zz
            return 0

        lax.fori_loop(0, CHUNK, zrow, 0)
        rbase = s * ROWS_PT
        for t in range(ROWS_PT // ZC):
            pltpu.sync_copy(prod_v.at[pl.ds(0, ZC)],
                            accum_sh.at[pl.ds(rbase + t * ZC, ZC)])
        plsc.subcore_barrier()
        ebase = wid * TPW

        ebase2 = wid * (TPW // 2)

        def start_idx_w(k, b):
            pltpu.async_copy(row_hbm.at[wid].at[pl.ds(k, 1)], RI[b], SR[b])
            pltpu.async_copy(col_hbm.at[wid].at[pl.ds(k, 1)], CI[b], SC[b])
            pltpu.async_copy(w_hbm.at[pl.ds(ebase2 + k * (CHUNK // 2), CHUNK // 2)],
                             WV[b], SW[b])

        def wait_r_start_gather(b):
            pltpu.make_async_copy(row_hbm.at[wid].at[pl.ds(0, 1)], RI[b], SR[b]).wait()
            pltpu.async_copy(vh_hbm.at[RI[b].at[0]], GV[b], SG[b])

        def scatter_wait():
            pltpu.make_async_copy(prod_v, accum_sh.at[cs_v.at[0]], sem_s).wait()

        def process(k, b):
            pltpu.make_async_copy(w_hbm.at[pl.ds(0, CHUNK // 2)], WV[b], SW[b]).wait()
            pltpu.make_async_copy(vh_hbm.at[pl.ds(0, CHUNK)], GV[b], SG[b]).wait()

            @pl.when(k > 0)
            def _():
                scatter_wait()

            def mrow(r, _):
                for q in range(H // 32):
                    vA = plsc.bitcast(GV[b][2 * r, pl.ds(LANE * q, LANE)], jnp.bfloat16)
                    vB = plsc.bitcast(GV[b][2 * r + 1, pl.ds(LANE * q, LANE)], jnp.bfloat16)
                    v0e, v1e = plsc.unpack(vA, format=plsc.PackFormat.INTERLEAVED)
                    v0o, v1o = plsc.unpack(vB, format=plsc.PackFormat.INTERLEAVED)
                    wA = plsc.bitcast(WV[b][r, pl.ds(32 * q, LANE)], jnp.bfloat16)
                    wB = plsc.bitcast(WV[b][r, pl.ds(32 * q + LANE, LANE)], jnp.bfloat16)
                    wAe, wAo = plsc.unpack(wA, format=plsc.PackFormat.INTERLEAVED)
                    wBe, wBo = plsc.unpack(wB, format=plsc.PackFormat.INTERLEAVED)
                    prod_v[2 * r, pl.ds(32 * q, LANE)] = wAe * v0e
                    prod_v[2 * r, pl.ds(32 * q + LANE, LANE)] = wBe * v1e
                    prod_v[2 * r + 1, pl.ds(32 * q, LANE)] = wAo * v0o
                    prod_v[2 * r + 1, pl.ds(32 * q + LANE, LANE)] = wBo * v1o
                return 0

            lax.fori_loop(0, CHUNK // 2, mrow, 0)
            pltpu.make_async_copy(col_hbm.at[wid].at[pl.ds(0, 1)], CI[b], SC[b]).wait()
            for t in range(CHUNK // LANE):
                cs_v[0, pl.ds(LANE * t, LANE)] = CI[b][0, pl.ds(LANE * t, LANE)]
            pltpu.async_copy(prod_v, accum_sh.at[cs_v.at[0]], sem_s, add=True)

        start_idx_w(0, 0)
        start_idx_w(1, 1)
        wait_r_start_gather(0)

        def body2(j, _):
            k = 2 * j
            wait_r_start_gather(1)
            process(k, 0)

            @pl.when(k + 2 < CPT)
            def _():
                start_idx_w(k + 2, 0)
            process(k + 1, 1)

            @pl.when(k + 3 < CPT)
            def _():
                start_idx_w(k + 3, 1)

            @pl.when(k + 2 < CPT)
            def _():
                wait_r_start_gather(0)
            return 0

        lax.fori_loop(0, CPT // 2, body2, 0)
        scatter_wait()
        plsc.subcore_barrier()
        pltpu.sync_copy(accum_sh.at[pl.ds(rbase, ROWS_PT)],
                        out_hbm.at[c, pl.ds(rbase, ROWS_PT)])

    k = pl.kernel(
        body,
        out_type=jax.ShapeDtypeStruct((NC, N, H), jnp.float32),
        mesh=mesh,
        compiler_params=pltpu.CompilerParams(needs_layout_passes=False,
                                             use_tc_tiling_on_sc=False),
        scratch_types=[
            pltpu.VMEM_SHARED((N, H), jnp.float32),
            pltpu.VMEM((1, CHUNK), jnp.int32),
            pltpu.VMEM((1, CHUNK), jnp.int32),
            pltpu.VMEM((1, CHUNK), jnp.int32),
            pltpu.VMEM((1, CHUNK), jnp.int32),
            pltpu.VMEM((CHUNK // 2, H), jnp.int32),
            pltpu.VMEM((CHUNK // 2, H), jnp.int32),
            pltpu.VMEM((CHUNK, H // 2), jnp.int32),
            pltpu.VMEM((CHUNK, H // 2), jnp.int32),
            pltpu.VMEM((CHUNK, H), jnp.float32),
            pltpu.VMEM((1, CHUNK), jnp.int32),
        ] + [pltpu.SemaphoreType.DMA] * 9,
    )
    return k(w_e, vh, rowC, colC)


def _update_call(part, v, w1t, b1, w2t, b2, lo, hi, N, H):
    BN = _largest_div(N, 1024, mult=8)

    def body(p_ref, v_ref, w1_ref, b1_ref, w2_ref, b2_ref, lo_ref, hi_ref,
             vn_ref, vh_ref):
        out = p_ref[0] + p_ref[1]
        h = _softplus(jnp.dot(out, w1_ref[...], preferred_element_type=jnp.float32)
                      + b1_ref[...]) - LN2
        upd = jnp.dot(h, w2_ref[...], preferred_element_type=jnp.float32) + b2_ref[...]
        vn = v_ref[...] + upd
        vn_ref[...] = vn
        vh_ref[...] = _pack_pair(
            jnp.dot(vn, lo_ref[...], preferred_element_type=jnp.float32),
            jnp.dot(vn, hi_ref[...], preferred_element_type=jnp.float32))

    grid = (N // BN,)
    return pl.pallas_call(
        body,
        grid=grid,
        in_specs=[
            pl.BlockSpec((2, BN, H), lambda b: (0, b, 0)),
            pl.BlockSpec((BN, H), lambda b: (b, 0)),
            pl.BlockSpec((H, H), lambda b: (0, 0)),
            pl.BlockSpec((1, H), lambda b: (0, 0)),
            pl.BlockSpec((H, H), lambda b: (0, 0)),
            pl.BlockSpec((1, H), lambda b: (0, 0)),
            pl.BlockSpec((H, H // 2), lambda b: (0, 0)),
            pl.BlockSpec((H, H // 2), lambda b: (0, 0)),
        ],
        out_specs=[
            pl.BlockSpec((BN, H), lambda b: (b, 0)),
            pl.BlockSpec((BN, H // 2), lambda b: (b, 0)),
        ],
        out_shape=[
            jax.ShapeDtypeStruct((N, H), jnp.float32),
            jax.ShapeDtypeStruct((N, H // 2), jnp.int32),
        ],
    )(part, v, w1t, b1, w2t, b2, lo, hi)


def _vh0_call(v, lo, hi, N, H):
    BN = _largest_div(N, 1024, mult=8)

    def body(v_ref, lo_ref, hi_ref, vh_ref):
        vh_ref[...] = _pack_pair(
            jnp.dot(v_ref[...], lo_ref[...], preferred_element_type=jnp.float32),
            jnp.dot(v_ref[...], hi_ref[...], preferred_element_type=jnp.float32))

    return pl.pallas_call(
        body,
        grid=(N // BN,),
        in_specs=[
            pl.BlockSpec((BN, H), lambda b: (b, 0)),
            pl.BlockSpec((H, H // 2), lambda b: (0, 0)),
            pl.BlockSpec((H, H // 2), lambda b: (0, 0)),
        ],
        out_specs=pl.BlockSpec((BN, H // 2), lambda b: (b, 0)),
        out_shape=jax.ShapeDtypeStruct((N, H // 2), jnp.int32),
    )(v, lo, hi)


def kernel(v, pos, edges, offsets_real, lin_w, mlp_w1, mlp_b1, mlp_w2, mlp_b2,
           v_w1, v_b1, v_w2, v_b2):
    N, H = v.shape
    L, FLT, G = mlp_w1.shape
    E = edges.shape[1]
    assert N % NS == 0 and H % LANE == 0

    CPT = -(-E // (NW * CHUNK))
    CPT += CPT % 2
    TPW = CPT * CHUNK
    E_PAD = NW * TPW
    GRP = TPW // LANE
    GP = -(-G // 8) * 8

    f32 = jnp.float32
    row = jnp.pad(edges[0], (0, E_PAD - E)).astype(jnp.int32)
    col = jnp.pad(edges[1], (0, E_PAD - E)).astype(jnp.int32)
    offp = jnp.pad(offsets_real, ((0, E_PAD - E), (0, 0))).astype(f32)
    row3 = row.reshape(NW, GRP, LANE)
    col3 = col.reshape(NW, GRP, LANE)
    ox3 = offp[:, 0].reshape(NW, GRP, LANE)
    oy3 = offp[:, 1].reshape(NW, GRP, LANE)
    oz3 = offp[:, 2].reshape(NW, GRP, LANE)
    px = pos[:, 0].astype(f32)
    py = pos[:, 1].astype(f32)
    pz = pos[:, 2].astype(f32)

    d2 = _d2_call(row3, col3, ox3, oy3, oz3, px, py, pz)
    d2f = d2.reshape(E_PAD)
    d2e = d2f[0::2].reshape(E_PAD // 512, 256)
    d2o = d2f[1::2].reshape(E_PAD // 512, 256)

    P2 = _interleave_perm(H)
    w1t = jnp.pad(jnp.swapaxes(mlp_w1, 1, 2), ((0, 0), (0, GP - G), (0, 0))).astype(f32)
    b1r = mlp_b1.reshape(L, 1, FLT).astype(f32)
    w2t = jnp.swapaxes(mlp_w2, 1, 2).astype(f32)
    b2r = mlp_b2.reshape(L, 1, FLT).astype(f32)
    W_all = _filters_call(d2e, d2o, w1t, b1r, w2t, b2r, E, L, G, GP, H, E_PAD)

    rowC = row.reshape(NW, CPT, CHUNK)
    colC = col.reshape(NW, CPT, CHUNK)

    v = v.astype(f32)

    def lohi(w):
        lp = jnp.swapaxes(w, 0, 1).astype(f32)[:, P2]
        return lp[:, 0::2], lp[:, 1::2]

    lo0, hi0 = lohi(lin_w[0])
    vh = _vh0_call(v, lo0, hi0, N, H)
    for l in range(L):
        part = _message_call(W_all[l], vh, rowC, colC, N, H, CPT, TPW)
        lo, hi = lohi(lin_w[(l + 1) % L])
        v, vh = _update_call(part, v,
                             jnp.swapaxes(v_w1[l], 0, 1).astype(f32),
                             v_b1[l].reshape(1, H).astype(f32),
                             jnp.swapaxes(v_w2[l], 0, 1).astype(f32),
                             v_b2[l].reshape(1, H).astype(f32),
                             lo, hi, N, H)
    return v


# speedup vs baseline: 1.8527x; 1.3702x over previous
import math
import numpy as np
import jax
import jax.numpy as jnp
from jax import lax
from jax.experimental import pallas as pl
from jax.experimental.pallas import tpu as pltpu
from jax.experimental.pallas import tpu_sc as plsc

CUTOFF = 5.0
LN2 = math.log(2.0)
NC = 2
NS = 16
NW = NC * NS
CHUNK = 128
LANE = 16


def _softplus(x):
    return jnp.maximum(x, 0.0) + jnp.log(1.0 + jnp.exp(-jnp.abs(x)))


def _pack_pair(a, b):
    pa = lax.bitcast_convert_type(a.astype(jnp.bfloat16), jnp.uint16).astype(jnp.uint32)
    pb = lax.bitcast_convert_type(b.astype(jnp.bfloat16), jnp.uint16).astype(jnp.uint32)
    return lax.bitcast_convert_type(pa | (pb << 16), jnp.int32)


def _interleave_perm(H):
    P = np.empty(H, np.int32)
    for m in range(H // 32):
        for t in range(16):
            P[32 * m + 2 * t] = 32 * m + t
            P[32 * m + 2 * t + 1] = 32 * m + 16 + t
    return P


def _largest_div(n, cap, mult=1):
    for d in range(min(n, cap), 0, -1):
        if n % d == 0 and d % mult == 0:
            return d
    return 1


def _d2_call(row3, col3, ox3, oy3, oz3, px, py, pz):
    NWv, GRP, _ = row3.shape
    mesh = plsc.VectorSubcoreMesh(core_axis_name="c", subcore_axis_name="s")

    def body(row_hbm, col_hbm, ox_hbm, oy_hbm, oz_hbm, px_hbm, py_hbm, pz_hbm,
             d2_hbm, px_v, py_v, pz_v, row_v, col_v, ox_v, oy_v, oz_v, d2_v):
        c = lax.axis_index("c")
        s = lax.axis_index("s")
        wid = s * NC + c
        pltpu.sync_copy(px_hbm, px_v)
        pltpu.sync_copy(py_hbm, py_v)
        pltpu.sync_copy(pz_hbm, pz_v)
        pltpu.sync_copy(row_hbm.at[wid], row_v)
        pltpu.sync_copy(col_hbm.at[wid], col_v)
        pltpu.sync_copy(ox_hbm.at[wid], ox_v)
        pltpu.sync_copy(oy_hbm.at[wid], oy_v)
        pltpu.sync_copy(oz_hbm.at[wid], oz_v)

        def step(i, _):
            r = row_v[i]
            cc = col_v[i]
            rx = plsc.load_gather(px_v, [r])
            ry = plsc.load_gather(py_v, [r])
            rz = plsc.load_gather(pz_v, [r])
            cx = plsc.load_gather(px_v, [cc])
            cy = plsc.load_gather(py_v, [cc])
            cz = plsc.load_gather(pz_v, [cc])
            dx = cx + ox_v[i] - rx
            dy = cy + oy_v[i] - ry
            dz = cz + oz_v[i] - rz
            d2_v[i] = dx * dx + dy * dy + dz * dz
            return 0

        lax.fori_loop(0, GRP, step, 0)
        pltpu.sync_copy(d2_v, d2_hbm.at[wid])

    Np = px.shape[0]
    k = pl.kernel(
        body,
        out_type=jax.ShapeDtypeStruct((NWv, GRP, LANE), jnp.float32),
        mesh=mesh,
        compiler_params=pltpu.CompilerParams(needs_layout_passes=False,
                                             use_tc_tiling_on_sc=False),
        scratch_types=[
            pltpu.VMEM((Np,), jnp.float32),
            pltpu.VMEM((Np,), jnp.float32),
            pltpu.VMEM((Np,), jnp.float32),
            pltpu.VMEM((GRP, LANE), jnp.int32),
            pltpu.VMEM((GRP, LANE), jnp.int32),
            pltpu.VMEM((GRP, LANE), jnp.float32),
            pltpu.VMEM((GRP, LANE), jnp.float32),
            pltpu.VMEM((GRP, LANE), jnp.float32),
            pltpu.VMEM((GRP, LANE), jnp.float32),
        ],
    )
    return k(row3, col3, ox3, oy3, oz3, px, py, pz)


def _filters_call(d2e, d2o, w1t, b1r, w2t, b2r, E, G, GP, H, E_PAD):
    BE = 2048
    nblk = E_PAD // 2 // BE

    def half(d2, w1, b1, w2, b2, base, step, coeff):
        dist = jnp.sqrt(d2)
        offs = lax.broadcasted_iota(jnp.int32, (BE, GP), 1).astype(jnp.float32) * step
        emb = jnp.exp(coeff * (dist[:, None] - offs) ** 2)
        h1 = jnp.dot(emb, w1, preferred_element_type=jnp.float32)
        h1 = _softplus(h1 + b1) - LN2
        Wf = jnp.dot(h1, w2, preferred_element_type=jnp.float32) + b2
        Cc = 0.5 * (jnp.cos(dist * (math.pi / CUTOFF)) + 1.0)
        eidx = base + 2 * lax.broadcasted_iota(jnp.int32, (BE,), 0)
        Cc = jnp.where(eidx < E, Cc, 0.0)
        return Wf * Cc[:, None]

    def body(d2e_ref, d2o_ref, w1t_ref, b1_ref, w2t_ref, b2_ref, wout_ref):
        b = pl.program_id(0)
        step = CUTOFF / (G - 1)
        coeff = -0.5 / (step * step)
        w1 = w1t_ref[...]
        b1 = b1_ref[...]
        w2 = w2t_ref[...]
        b2 = b2_ref[...]
        We = half(d2e_ref[...].reshape(BE), w1, b1, w2, b2, b * 2 * BE, step, coeff)
        Wo = half(d2o_ref[...].reshape(BE), w1, b1, w2, b2, b * 2 * BE + 1, step, coeff)
        wout_ref[...] = _pack_pair(We, Wo)

    return pl.pallas_call(
        body,
        grid=(nblk,),
        in_specs=[
            pl.BlockSpec((BE // 256, 256), lambda b: (b, 0)),
            pl.BlockSpec((BE // 256, 256), lambda b: (b, 0)),
            pl.BlockSpec((GP, H), lambda b: (0, 0)),
            pl.BlockSpec((1, H), lambda b: (0, 0)),
            pl.BlockSpec((H, H), lambda b: (0, 0)),
            pl.BlockSpec((1, H), lambda b: (0, 0)),
        ],
        out_specs=pl.BlockSpec((BE, H), lambda b: (b, 0)),
        out_shape=jax.ShapeDtypeStruct((E_PAD // 2, H), jnp.int32),
    )(d2e, d2o, w1t, b1r, w2t, b2r)


def _message_call(w_e, vh, rowC, colC, N, H, CPT, TPW):
    mesh = plsc.VectorSubcoreMesh(core_axis_name="c", subcore_axis_name="s")
    ROWS_PT = N // NS
    ZC = _largest_div(ROWS_PT, CHUNK)

    def body(w_hbm, vh_hbm, row_hbm, col_hbm, out_hbm, accum_sh,
             ri0, ri1, ci0, ci1, wv0, wv1, gv0, gv1, prod_v, cs_v,
             sr0, sr1, sc0, sc1, sw0, sw1, sg0, sg1, sem_s):
        c = lax.axis_index("c")
        s = lax.axis_index("s")
        wid = s * NC + c
        RI = (ri0, ri1)
        CI = (ci0, ci1)
        WV = (wv0, wv1)
        GV = (gv0, gv1)
        SR = (sr0, sr1)
        SC = (sc0, sc1)
        SW = (sw0, sw1)
        SG = (sg0, sg1)
        zz = jnp.zeros((LANE,), jnp.float32)

        def zrow(i, _):
            for j in range(H // LANE):
                prod_v[i, pl.ds(j * LANE, LANE)] = zz
            return 0

        lax.fori_loop(0, CHUNK, zrow, 0)
        rbase = s * ROWS_PT
        for t in range(ROWS_PT // ZC):
            pltpu.sync_copy(prod_v.at[pl.ds(0, ZC)],
                            accum_sh.at[pl.ds(rbase + t * ZC, ZC)])
        plsc.subcore_barrier()
        ebase = wid * TPW

        ebase2 = wid * (TPW // 2)

        def start_idx_w(k, b):
            pltpu.async_copy(row_hbm.at[wid].at[pl.ds(k, 1)], RI[b], SR[b])
            pltpu.async_copy(col_hbm.at[wid].at[pl.ds(k, 1)], CI[b], SC[b])
            pltpu.async_copy(w_hbm.at[pl.ds(ebase2 + k * (CHUNK // 2), CHUNK // 2)],
                             WV[b], SW[b])

        def wait_r_start_gather(b):
            pltpu.make_async_copy(row_hbm.at[wid].at[pl.ds(0, 1)], RI[b], SR[b]).wait()
            pltpu.async_copy(vh_hbm.at[RI[b].at[0]], GV[b], SG[b])

        def scatter_wait():
            pltpu.make_async_copy(prod_v, accum_sh.at[cs_v.at[0]], sem_s).wait()

        def process(k, b):
            pltpu.make_async_copy(w_hbm.at[pl.ds(0, CHUNK // 2)], WV[b], SW[b]).wait()
            pltpu.make_async_copy(vh_hbm.at[pl.ds(0, CHUNK)], GV[b], SG[b]).wait()

            @pl.when(k > 0)
            def _():
                scatter_wait()

            def mrow(r, _):
                for q in range(H // 32):
                    vA = plsc.bitcast(GV[b][2 * r, pl.ds(LANE * q, LANE)], jnp.bfloat16)
                    vB = plsc.bitcast(GV[b][2 * r + 1, pl.ds(LANE * q, LANE)], jnp.bfloat16)
                    v0e, v1e = plsc.unpack(vA, format=plsc.PackFormat.INTERLEAVED)
                    v0o, v1o = plsc.unpack(vB, format=plsc.PackFormat.INTERLEAVED)
                    wA = plsc.bitcast(WV[b][r, pl.ds(32 * q, LANE)], jnp.bfloat16)
                    wB = plsc.bitcast(WV[b][r, pl.ds(32 * q + LANE, LANE)], jnp.bfloat16)
                    wAe, wAo = plsc.unpack(wA, format=plsc.PackFormat.INTERLEAVED)
                    wBe, wBo = plsc.unpack(wB, format=plsc.PackFormat.INTERLEAVED)
                    prod_v[2 * r, pl.ds(32 * q, LANE)] = wAe * v0e
                    prod_v[2 * r, pl.ds(32 * q + LANE, LANE)] = wBe * v1e
                    prod_v[2 * r + 1, pl.ds(32 * q, LANE)] = wAo * v0o
                    prod_v[2 * r + 1, pl.ds(32 * q + LANE, LANE)] = wBo * v1o
                return 0

            lax.fori_loop(0, CHUNK // 2, mrow, 0)
            pltpu.make_async_copy(col_hbm.at[wid].at[pl.ds(0, 1)], CI[b], SC[b]).wait()
            for t in range(CHUNK // LANE):
                cs_v[0, pl.ds(LANE * t, LANE)] = CI[b][0, pl.ds(LANE * t, LANE)]
            pltpu.async_copy(prod_v, accum_sh.at[cs_v.at[0]], sem_s, add=True)

        start_idx_w(0, 0)
        start_idx_w(1, 1)
        wait_r_start_gather(0)

        def body2(j, _):
            k = 2 * j
            wait_r_start_gather(1)
            process(k, 0)

            @pl.when(k + 2 < CPT)
            def _():
                start_idx_w(k + 2, 0)
            process(k + 1, 1)

            @pl.when(k + 3 < CPT)
            def _():
                start_idx_w(k + 3, 1)

            @pl.when(k + 2 < CPT)
            def _():
                wait_r_start_gather(0)
            return 0

        lax.fori_loop(0, CPT // 2, body2, 0)
        scatter_wait()
        plsc.subcore_barrier()
        pltpu.sync_copy(accum_sh.at[pl.ds(rbase, ROWS_PT)],
                        out_hbm.at[c, pl.ds(rbase, ROWS_PT)])

    k = pl.kernel(
        body,
        out_type=jax.ShapeDtypeStruct((NC, N, H), jnp.float32),
        mesh=mesh,
        compiler_params=pltpu.CompilerParams(needs_layout_passes=False,
                                             use_tc_tiling_on_sc=False),
        scratch_types=[
            pltpu.VMEM_SHARED((N, H), jnp.float32),
            pltpu.VMEM((1, CHUNK), jnp.int32),
            pltpu.VMEM((1, CHUNK), jnp.int32),
            pltpu.VMEM((1, CHUNK), jnp.int32),
            pltpu.VMEM((1, CHUNK), jnp.int32),
            pltpu.VMEM((CHUNK // 2, H), jnp.int32),
            pltpu.VMEM((CHUNK // 2, H), jnp.int32),
            pltpu.VMEM((CHUNK, H // 2), jnp.int32),
            pltpu.VMEM((CHUNK, H // 2), jnp.int32),
            pltpu.VMEM((CHUNK, H), jnp.float32),
            pltpu.VMEM((1, CHUNK), jnp.int32),
        ] + [pltpu.SemaphoreType.DMA] * 9,
    )
    return k(w_e, vh, rowC, colC)


def _update_call(part, v, w1t, b1, w2t, b2, lo, hi, N, H):
    BN = _largest_div(N, 1024, mult=8)

    def body(p_ref, v_ref, w1_ref, b1_ref, w2_ref, b2_ref, lo_ref, hi_ref,
             vn_ref, vh_ref):
        out = p_ref[0] + p_ref[1]
        h = _softplus(jnp.dot(out, w1_ref[...], preferred_element_type=jnp.float32)
                      + b1_ref[...]) - LN2
        upd = jnp.dot(h, w2_ref[...], preferred_element_type=jnp.float32) + b2_ref[...]
        vn = v_ref[...] + upd
        vn_ref[...] = vn
        vh_ref[...] = _pack_pair(
            jnp.dot(vn, lo_ref[...], preferred_element_type=jnp.float32),
            jnp.dot(vn, hi_ref[...], preferred_element_type=jnp.float32))

    grid = (N // BN,)
    return pl.pallas_call(
        body,
        grid=grid,
        in_specs=[
            pl.BlockSpec((2, BN, H), lambda b: (0, b, 0)),
            pl.BlockSpec((BN, H), lambda b: (b, 0)),
            pl.BlockSpec((H, H), lambda b: (0, 0)),
            pl.BlockSpec((1, H), lambda b: (0, 0)),
            pl.BlockSpec((H, H), lambda b: (0, 0)),
            pl.BlockSpec((1, H), lambda b: (0, 0)),
            pl.BlockSpec((H, H // 2), lambda b: (0, 0)),
            pl.BlockSpec((H, H // 2), lambda b: (0, 0)),
        ],
        out_specs=[
            pl.BlockSpec((BN, H), lambda b: (b, 0)),
            pl.BlockSpec((BN, H // 2), lambda b: (b, 0)),
        ],
        out_shape=[
            jax.ShapeDtypeStruct((N, H), jnp.float32),
            jax.ShapeDtypeStruct((N, H // 2), jnp.int32),
        ],
    )(part, v, w1t, b1, w2t, b2, lo, hi)


def _vh0_call(v, lo, hi, N, H):
    BN = _largest_div(N, 1024, mult=8)

    def body(v_ref, lo_ref, hi_ref, vh_ref):
        vh_ref[...] = _pack_pair(
            jnp.dot(v_ref[...], lo_ref[...], preferred_element_type=jnp.float32),
            jnp.dot(v_ref[...], hi_ref[...], preferred_element_type=jnp.float32))

    return pl.pallas_call(
        body,
        grid=(N // BN,),
        in_specs=[
            pl.BlockSpec((BN, H), lambda b: (b, 0)),
            pl.BlockSpec((H, H // 2), lambda b: (0, 0)),
            pl.BlockSpec((H, H // 2), lambda b: (0, 0)),
        ],
        out_specs=pl.BlockSpec((BN, H // 2), lambda b: (b, 0)),
        out_shape=jax.ShapeDtypeStruct((N, H // 2), jnp.int32),
    )(v, lo, hi)


def kernel(v, pos, edges, offsets_real, lin_w, mlp_w1, mlp_b1, mlp_w2, mlp_b2,
           v_w1, v_b1, v_w2, v_b2):
    N, H = v.shape
    L, FLT, G = mlp_w1.shape
    E = edges.shape[1]
    assert N % NS == 0 and H % LANE == 0

    CPT = -(-E // (NW * CHUNK))
    CPT += CPT % 2
    TPW = CPT * CHUNK
    E_PAD = NW * TPW
    GRP = TPW // LANE
    GP = -(-G // 8) * 8

    f32 = jnp.float32
    row = jnp.pad(edges[0], (0, E_PAD - E)).astype(jnp.int32)
    col = jnp.pad(edges[1], (0, E_PAD - E)).astype(jnp.int32)
    offp = jnp.pad(offsets_real, ((0, E_PAD - E), (0, 0))).astype(f32)
    row3 = row.reshape(NW, GRP, LANE)
    col3 = col.reshape(NW, GRP, LANE)
    ox3 = offp[:, 0].reshape(NW, GRP, LANE)
    oy3 = offp[:, 1].reshape(NW, GRP, LANE)
    oz3 = offp[:, 2].reshape(NW, GRP, LANE)
    px = pos[:, 0].astype(f32)
    py = pos[:, 1].astype(f32)
    pz = pos[:, 2].astype(f32)

    d2 = _d2_call(row3, col3, ox3, oy3, oz3, px, py, pz)
    d2f = d2.reshape(E_PAD)
    d2e = d2f[0::2].reshape(E_PAD // 512, 256)
    d2o = d2f[1::2].reshape(E_PAD // 512, 256)

    P2 = _interleave_perm(H)
    w1t = jnp.pad(jnp.swapaxes(mlp_w1, 1, 2), ((0, 0), (0, GP - G), (0, 0))).astype(f32)
    b1r = mlp_b1.reshape(L, 1, FLT).astype(f32)
    w2t = jnp.swapaxes(mlp_w2, 1, 2).astype(f32)
    b2r = mlp_b2.reshape(L, 1, FLT).astype(f32)

    rowC = row.reshape(NW, CPT, CHUNK)
    colC = col.reshape(NW, CPT, CHUNK)

    v = v.astype(f32)

    def lohi(w):
        lp = jnp.swapaxes(w, 0, 1).astype(f32)[:, P2]
        return lp[:, 0::2], lp[:, 1::2]

    lo0, hi0 = lohi(lin_w[0])
    vh = _vh0_call(v, lo0, hi0, N, H)
    for l in range(L):
        W_l = _filters_call(d2e, d2o, w1t[l], b1r[l], w2t[l], b2r[l],
                            E, G, GP, H, E_PAD)
        part = _message_call(W_l, vh, rowC, colC, N, H, CPT, TPW)
        lo, hi = lohi(lin_w[(l + 1) % L])
        v, vh = _update_call(part, v,
                             jnp.swapaxes(v_w1[l], 0, 1).astype(f32),
                             v_b1[l].reshape(1, H).astype(f32),
                             jnp.swapaxes(v_w2[l], 0, 1).astype(f32),
                             v_b2[l].reshape(1, H).astype(f32),
                             lo, hi, N, H)
    return v


# speedup vs baseline: 2.1312x; 1.1503x over previous
import math
import numpy as np
import jax
import jax.numpy as jnp
from jax import lax
from jax.experimental import pallas as pl
from jax.experimental.pallas import tpu as pltpu
from jax.experimental.pallas import tpu_sc as plsc

CUTOFF = 5.0
LN2 = math.log(2.0)
NC = 2
NS = 16
NW = NC * NS
CHUNK = 128
LANE = 16


def _softplus(x):
    return jnp.maximum(x, 0.0) + jnp.log(1.0 + jnp.exp(-jnp.abs(x)))


def _pack_pair(a, b):
    pa = lax.bitcast_convert_type(a.astype(jnp.bfloat16), jnp.uint16).astype(jnp.uint32)
    pb = lax.bitcast_convert_type(b.astype(jnp.bfloat16), jnp.uint16).astype(jnp.uint32)
    return lax.bitcast_convert_type(pa | (pb << 16), jnp.int32)


def _interleave_perm(H):
    P = np.empty(H, np.int32)
    for m in range(H // 32):
        for t in range(16):
            P[32 * m + 2 * t] = 32 * m + t
            P[32 * m + 2 * t + 1] = 32 * m + 16 + t
    return P


def _largest_div(n, cap, mult=1):
    for d in range(min(n, cap), 0, -1):
        if n % d == 0 and d % mult == 0:
            return d
    return 1


def _d2_call(row3, col3, ox3, oy3, oz3, px, py, pz):
    NWv, GRP, _ = row3.shape
    mesh = plsc.VectorSubcoreMesh(core_axis_name="c", subcore_axis_name="s")

    def body(row_hbm, col_hbm, ox_hbm, oy_hbm, oz_hbm, px_hbm, py_hbm, pz_hbm,
             d2_hbm, px_v, py_v, pz_v, row_v, col_v, ox_v, oy_v, oz_v, d2_v):
        c = lax.axis_index("c")
        s = lax.axis_index("s")
        wid = s * NC + c
        pltpu.sync_copy(px_hbm, px_v)
        pltpu.sync_copy(py_hbm, py_v)
        pltpu.sync_copy(pz_hbm, pz_v)
        pltpu.sync_copy(row_hbm.at[wid], row_v)
        pltpu.sync_copy(col_hbm.at[wid], col_v)
        pltpu.sync_copy(ox_hbm.at[wid], ox_v)
        pltpu.sync_copy(oy_hbm.at[wid], oy_v)
        pltpu.sync_copy(oz_hbm.at[wid], oz_v)

        def step(i, _):
            r = row_v[i]
            cc = col_v[i]
            rx = plsc.load_gather(px_v, [r])
            ry = plsc.load_gather(py_v, [r])
            rz = plsc.load_gather(pz_v, [r])
            cx = plsc.load_gather(px_v, [cc])
            cy = plsc.load_gather(py_v, [cc])
            cz = plsc.load_gather(pz_v, [cc])
            dx = cx + ox_v[i] - rx
            dy = cy + oy_v[i] - ry
            dz = cz + oz_v[i] - rz
            d2_v[i] = dx * dx + dy * dy + dz * dz
            return 0

        lax.fori_loop(0, GRP, step, 0)
        pltpu.sync_copy(d2_v, d2_hbm.at[wid])

    Np = px.shape[0]
    k = pl.kernel(
        body,
        out_type=jax.ShapeDtypeStruct((NWv, GRP, LANE), jnp.float32),
        mesh=mesh,
        compiler_params=pltpu.CompilerParams(needs_layout_passes=False,
                                             use_tc_tiling_on_sc=False),
        scratch_types=[
            pltpu.VMEM((Np,), jnp.float32),
            pltpu.VMEM((Np,), jnp.float32),
            pltpu.VMEM((Np,), jnp.float32),
            pltpu.VMEM((GRP, LANE), jnp.int32),
            pltpu.VMEM((GRP, LANE), jnp.int32),
            pltpu.VMEM((GRP, LANE), jnp.float32),
            pltpu.VMEM((GRP, LANE), jnp.float32),
            pltpu.VMEM((GRP, LANE), jnp.float32),
            pltpu.VMEM((GRP, LANE), jnp.float32),
        ],
    )
    return k(row3, col3, ox3, oy3, oz3, px, py, pz)


def _filters_call(d2e, d2o, w1t, b1r, w2t, b2r, E, G, GP, H, E_PAD):
    BE = 2048
    nblk = E_PAD // 2 // BE

    def half(d2, w1, b1, w2, b2, base, step, coeff):
        dist = jnp.sqrt(d2)
        offs = lax.broadcasted_iota(jnp.int32, (BE, GP), 1).astype(jnp.float32) * step
        emb = jnp.exp(coeff * (dist[:, None] - offs) ** 2)
        h1 = jnp.dot(emb, w1, preferred_element_type=jnp.float32)
        h1 = _softplus(h1 + b1) - LN2
        Wf = jnp.dot(h1, w2, preferred_element_type=jnp.float32) + b2
        Cc = 0.5 * (jnp.cos(dist * (math.pi / CUTOFF)) + 1.0)
        eidx = base + 2 * lax.broadcasted_iota(jnp.int32, (BE,), 0)
        Cc = jnp.where(eidx < E, Cc, 0.0)
        return Wf * Cc[:, None]

    def body(d2e_ref, d2o_ref, w1t_ref, b1_ref, w2t_ref, b2_ref, wout_ref):
        b = pl.program_id(0)
        step = CUTOFF / (G - 1)
        coeff = -0.5 / (step * step)
        w1 = w1t_ref[...]
        b1 = b1_ref[...]
        w2 = w2t_ref[...]
        b2 = b2_ref[...]
        We = half(d2e_ref[...].reshape(BE), w1, b1, w2, b2, b * 2 * BE, step, coeff)
        Wo = half(d2o_ref[...].reshape(BE), w1, b1, w2, b2, b * 2 * BE + 1, step, coeff)
        wout_ref[...] = _pack_pair(We, Wo)

    return pl.pallas_call(
        body,
        grid=(nblk,),
        in_specs=[
            pl.BlockSpec((BE // 256, 256), lambda b: (b, 0)),
            pl.BlockSpec((BE // 256, 256), lambda b: (b, 0)),
            pl.BlockSpec((GP, H), lambda b: (0, 0)),
            pl.BlockSpec((1, H), lambda b: (0, 0)),
            pl.BlockSpec((H, H), lambda b: (0, 0)),
            pl.BlockSpec((1, H), lambda b: (0, 0)),
        ],
        out_specs=pl.BlockSpec((BE, H), lambda b: (b, 0)),
        out_shape=jax.ShapeDtypeStruct((E_PAD // 2, H), jnp.int32),
    )(d2e, d2o, w1t, b1r, w2t, b2r)


def _message_call(w_e, vh, rowC, colC, N, H, CPT, TPW):
    mesh = plsc.VectorSubcoreMesh(core_axis_name="c", subcore_axis_name="s")
    ROWS_PT = N // NS
    ZC = _largest_div(ROWS_PT, CHUNK)

    def body(w_hbm, vh_hbm, row_hbm, col_hbm, out_hbm, accum_sh,
             ri0, ri1, ci0, ci1, wv0, wv1, gv0, gv1, prod_v, cs_v,
             sr0, sr1, sc0, sc1, sw0, sw1, sg0, sg1, sem_s):
        c = lax.axis_index("c")
        s = lax.axis_index("s")
        wid = s * NC + c
        RI = (ri0, ri1)
        CI = (ci0, ci1)
        WV = (wv0, wv1)
        GV = (gv0, gv1)
        SR = (sr0, sr1)
        SC = (sc0, sc1)
        SW = (sw0, sw1)
        SG = (sg0, sg1)
        zz = jnp.zeros((LANE,), jnp.float32)

        @plsc.parallel_loop(0, CHUNK, 1, unroll=4)
        def zrow(i):
            for j in range(H // LANE):
                prod_v[i, pl.ds(j * LANE, LANE)] = zz
        rbase = s * ROWS_PT
        for t in range(ROWS_PT // ZC):
            pltpu.sync_copy(prod_v.at[pl.ds(0, ZC)],
                            accum_sh.at[pl.ds(rbase + t * ZC, ZC)])
        plsc.subcore_barrier()
        ebase = wid * TPW

        ebase2 = wid * (TPW // 2)

        def start_idx_w(k, b):
            pltpu.async_copy(row_hbm.at[wid].at[pl.ds(k, 1)], RI[b], SR[b])
            pltpu.async_copy(col_hbm.at[wid].at[pl.ds(k, 1)], CI[b], SC[b])
            pltpu.async_copy(w_hbm.at[pl.ds(ebase2 + k * (CHUNK // 2), CHUNK // 2)],
                             WV[b], SW[b])

        def wait_r_start_gather(b):
            pltpu.make_async_copy(row_hbm.at[wid].at[pl.ds(0, 1)], RI[b], SR[b]).wait()
            pltpu.async_copy(vh_hbm.at[RI[b].at[0]], GV[b], SG[b])

        def scatter_wait():
            pltpu.make_async_copy(prod_v, accum_sh.at[cs_v.at[0]], sem_s).wait()

        def process(k, b):
            pltpu.make_async_copy(w_hbm.at[pl.ds(0, CHUNK // 2)], WV[b], SW[b]).wait()
            pltpu.make_async_copy(vh_hbm.at[pl.ds(0, CHUNK)], GV[b], SG[b]).wait()

            @pl.when(k > 0)
            def _():
                scatter_wait()

            @plsc.parallel_loop(0, CHUNK // 2, 1, unroll=4)
            def mrow(r):
                for q in range(H // 32):
                    vA = plsc.bitcast(GV[b][2 * r, pl.ds(LANE * q, LANE)], jnp.bfloat16)
                    vB = plsc.bitcast(GV[b][2 * r + 1, pl.ds(LANE * q, LANE)], jnp.bfloat16)
                    v0e, v1e = plsc.unpack(vA, format=plsc.PackFormat.INTERLEAVED)
                    v0o, v1o = plsc.unpack(vB, format=plsc.PackFormat.INTERLEAVED)
                    wA = plsc.bitcast(WV[b][r, pl.ds(32 * q, LANE)], jnp.bfloat16)
                    wB = plsc.bitcast(WV[b][r, pl.ds(32 * q + LANE, LANE)], jnp.bfloat16)
                    wAe, wAo = plsc.unpack(wA, format=plsc.PackFormat.INTERLEAVED)
                    wBe, wBo = plsc.unpack(wB, format=plsc.PackFormat.INTERLEAVED)
                    prod_v[2 * r, pl.ds(32 * q, LANE)] = wAe * v0e
                    prod_v[2 * r, pl.ds(32 * q + LANE, LANE)] = wBe * v1e
                    prod_v[2 * r + 1, pl.ds(32 * q, LANE)] = wAo * v0o
                    prod_v[2 * r + 1, pl.ds(32 * q + LANE, LANE)] = wBo * v1o
            pltpu.make_async_copy(col_hbm.at[wid].at[pl.ds(0, 1)], CI[b], SC[b]).wait()
            for t in range(CHUNK // LANE):
                cs_v[0, pl.ds(LANE * t, LANE)] = CI[b][0, pl.ds(LANE * t, LANE)]
            pltpu.async_copy(prod_v, accum_sh.at[cs_v.at[0]], sem_s, add=True)

        start_idx_w(0, 0)
        start_idx_w(1, 1)
        wait_r_start_gather(0)

        def body2(j, _):
            k = 2 * j
            wait_r_start_gather(1)
            process(k, 0)

            @pl.when(k + 2 < CPT)
            def _():
                start_idx_w(k + 2, 0)
            process(k + 1, 1)

            @pl.when(k + 3 < CPT)
            def _():
                start_idx_w(k + 3, 1)

            @pl.when(k + 2 < CPT)
            def _():
                wait_r_start_gather(0)
            return 0

        lax.fori_loop(0, CPT // 2, body2, 0)
        scatter_wait()
        plsc.subcore_barrier()
        pltpu.sync_copy(accum_sh.at[pl.ds(rbase, ROWS_PT)],
                        out_hbm.at[c, pl.ds(rbase, ROWS_PT)])

    k = pl.kernel(
        body,
        out_type=jax.ShapeDtypeStruct((NC, N, H), jnp.float32),
        mesh=mesh,
        compiler_params=pltpu.CompilerParams(needs_layout_passes=False,
                                             use_tc_tiling_on_sc=False),
        scratch_types=[
            pltpu.VMEM_SHARED((N, H), jnp.float32),
            pltpu.VMEM((1, CHUNK), jnp.int32),
            pltpu.VMEM((1, CHUNK), jnp.int32),
            pltpu.VMEM((1, CHUNK), jnp.int32),
            pltpu.VMEM((1, CHUNK), jnp.int32),
            pltpu.VMEM((CHUNK // 2, H), jnp.int32),
            pltpu.VMEM((CHUNK // 2, H), jnp.int32),
            pltpu.VMEM((CHUNK, H // 2), jnp.int32),
            pltpu.VMEM((CHUNK, H // 2), jnp.int32),
            pltpu.VMEM((CHUNK, H), jnp.float32),
            pltpu.VMEM((1, CHUNK), jnp.int32),
        ] + [pltpu.SemaphoreType.DMA] * 9,
    )
    return k(w_e, vh, rowC, colC)


def _update_call(part, v, w1t, b1, w2t, b2, lo, hi, N, H):
    BN = _largest_div(N, 1024, mult=8)

    def body(p_ref, v_ref, w1_ref, b1_ref, w2_ref, b2_ref, lo_ref, hi_ref,
             vn_ref, vh_ref):
        out = p_ref[0] + p_ref[1]
        h = _softplus(jnp.dot(out, w1_ref[...], preferred_element_type=jnp.float32)
                      + b1_ref[...]) - LN2
        upd = jnp.dot(h, w2_ref[...], preferred_element_type=jnp.float32) + b2_ref[...]
        vn = v_ref[...] + upd
        vn_ref[...] = vn
        vh_ref[...] = _pack_pair(
            jnp.dot(vn, lo_ref[...], preferred_element_type=jnp.float32),
            jnp.dot(vn, hi_ref[...], preferred_element_type=jnp.float32))

    grid = (N // BN,)
    return pl.pallas_call(
        body,
        grid=grid,
        in_specs=[
            pl.BlockSpec((2, BN, H), lambda b: (0, b, 0)),
            pl.BlockSpec((BN, H), lambda b: (b, 0)),
            pl.BlockSpec((H, H), lambda b: (0, 0)),
            pl.BlockSpec((1, H), lambda b: (0, 0)),
            pl.BlockSpec((H, H), lambda b: (0, 0)),
            pl.BlockSpec((1, H), lambda b: (0, 0)),
            pl.BlockSpec((H, H // 2), lambda b: (0, 0)),
            pl.BlockSpec((H, H // 2), lambda b: (0, 0)),
        ],
        out_specs=[
            pl.BlockSpec((BN, H), lambda b: (b, 0)),
            pl.BlockSpec((BN, H // 2), lambda b: (b, 0)),
        ],
        out_shape=[
            jax.ShapeDtypeStruct((N, H), jnp.float32),
            jax.ShapeDtypeStruct((N, H // 2), jnp.int32),
        ],
    )(part, v, w1t, b1, w2t, b2, lo, hi)


def _vh0_call(v, lo, hi, N, H):
    BN = _largest_div(N, 1024, mult=8)

    def body(v_ref, lo_ref, hi_ref, vh_ref):
        vh_ref[...] = _pack_pair(
            jnp.dot(v_ref[...], lo_ref[...], preferred_element_type=jnp.float32),
            jnp.dot(v_ref[...], hi_ref[...], preferred_element_type=jnp.float32))

    return pl.pallas_call(
        body,
        grid=(N // BN,),
        in_specs=[
            pl.BlockSpec((BN, H), lambda b: (b, 0)),
            pl.BlockSpec((H, H // 2), lambda b: (0, 0)),
            pl.BlockSpec((H, H // 2), lambda b: (0, 0)),
        ],
        out_specs=pl.BlockSpec((BN, H // 2), lambda b: (b, 0)),
        out_shape=jax.ShapeDtypeStruct((N, H // 2), jnp.int32),
    )(v, lo, hi)


def kernel(v, pos, edges, offsets_real, lin_w, mlp_w1, mlp_b1, mlp_w2, mlp_b2,
           v_w1, v_b1, v_w2, v_b2):
    N, H = v.shape
    L, FLT, G = mlp_w1.shape
    E = edges.shape[1]
    assert N % NS == 0 and H % LANE == 0

    CPT = -(-E // (NW * CHUNK))
    CPT += CPT % 2
    TPW = CPT * CHUNK
    E_PAD = NW * TPW
    GRP = TPW // LANE
    GP = -(-G // 8) * 8

    f32 = jnp.float32
    row = jnp.pad(edges[0], (0, E_PAD - E)).astype(jnp.int32)
    col = jnp.pad(edges[1], (0, E_PAD - E)).astype(jnp.int32)
    offp = jnp.pad(offsets_real, ((0, E_PAD - E), (0, 0))).astype(f32)
    row3 = row.reshape(NW, GRP, LANE)
    col3 = col.reshape(NW, GRP, LANE)
    ox3 = offp[:, 0].reshape(NW, GRP, LANE)
    oy3 = offp[:, 1].reshape(NW, GRP, LANE)
    oz3 = offp[:, 2].reshape(NW, GRP, LANE)
    px = pos[:, 0].astype(f32)
    py = pos[:, 1].astype(f32)
    pz = pos[:, 2].astype(f32)

    d2 = _d2_call(row3, col3, ox3, oy3, oz3, px, py, pz)
    d2f = d2.reshape(E_PAD)
    d2e = d2f[0::2].reshape(E_PAD // 512, 256)
    d2o = d2f[1::2].reshape(E_PAD // 512, 256)

    P2 = _interleave_perm(H)
    w1t = jnp.pad(jnp.swapaxes(mlp_w1, 1, 2), ((0, 0), (0, GP - G), (0, 0))).astype(f32)
    b1r = mlp_b1.reshape(L, 1, FLT).astype(f32)
    w2t = jnp.swapaxes(mlp_w2, 1, 2).astype(f32)
    b2r = mlp_b2.reshape(L, 1, FLT).astype(f32)

    rowC = row.reshape(NW, CPT, CHUNK)
    colC = col.reshape(NW, CPT, CHUNK)

    v = v.astype(f32)

    def lohi(w):
        lp = jnp.swapaxes(w, 0, 1).astype(f32)[:, P2]
        return lp[:, 0::2], lp[:, 1::2]

    lo0, hi0 = lohi(lin_w[0])
    vh = _vh0_call(v, lo0, hi0, N, H)
    for l in range(L):
        W_l = _filters_call(d2e, d2o, w1t[l], b1r[l], w2t[l], b2r[l],
                            E, G, GP, H, E_PAD)
        part = _message_call(W_l, vh, rowC, colC, N, H, CPT, TPW)
        lo, hi = lohi(lin_w[(l + 1) % L])
        v, vh = _update_call(part, v,
                             jnp.swapaxes(v_w1[l], 0, 1).astype(f32),
                             v_b1[l].reshape(1, H).astype(f32),
                             jnp.swapaxes(v_w2[l], 0, 1).astype(f32),
                             v_b2[l].reshape(1, H).astype(f32),
                             lo, hi, N, H)
    return v


# speedup vs baseline: 2.4655x; 1.1569x over previous
import math
import numpy as np
import jax
import jax.numpy as jnp
from jax import lax
from jax.experimental import pallas as pl
from jax.experimental.pallas import tpu as pltpu
from jax.experimental.pallas import tpu_sc as plsc

CUTOFF = 5.0
LN2 = math.log(2.0)
NC = 2
NS = 16
NW = NC * NS
CHUNK = 128
LANE = 16


def _softplus(x):
    return jnp.maximum(x, 0.0) + jnp.log(1.0 + jnp.exp(-jnp.abs(x)))


def _pack_pair(a, b):
    pa = lax.bitcast_convert_type(a.astype(jnp.bfloat16), jnp.uint16).astype(jnp.uint32)
    pb = lax.bitcast_convert_type(b.astype(jnp.bfloat16), jnp.uint16).astype(jnp.uint32)
    return lax.bitcast_convert_type(pa | (pb << 16), jnp.int32)


def _interleave_perm(H):
    P = np.empty(H, np.int32)
    for m in range(H // 32):
        for t in range(16):
            P[32 * m + 2 * t] = 32 * m + t
            P[32 * m + 2 * t + 1] = 32 * m + 16 + t
    return P


def _largest_div(n, cap, mult=1):
    for d in range(min(n, cap), 0, -1):
        if n % d == 0 and d % mult == 0:
            return d
    return 1


def _d2_call(row3, col3, ox3, oy3, oz3, px, py, pz):
    NWv, GRP, _ = row3.shape
    mesh = plsc.VectorSubcoreMesh(core_axis_name="c", subcore_axis_name="s")

    def body(row_hbm, col_hbm, ox_hbm, oy_hbm, oz_hbm, px_hbm, py_hbm, pz_hbm,
             d2_hbm, px_v, py_v, pz_v, row_v, col_v, ox_v, oy_v, oz_v, d2_v):
        c = lax.axis_index("c")
        s = lax.axis_index("s")
        wid = s * NC + c
        pltpu.sync_copy(px_hbm, px_v)
        pltpu.sync_copy(py_hbm, py_v)
        pltpu.sync_copy(pz_hbm, pz_v)
        pltpu.sync_copy(row_hbm.at[wid], row_v)
        pltpu.sync_copy(col_hbm.at[wid], col_v)
        pltpu.sync_copy(ox_hbm.at[wid], ox_v)
        pltpu.sync_copy(oy_hbm.at[wid], oy_v)
        pltpu.sync_copy(oz_hbm.at[wid], oz_v)

        def step(i, _):
            r = row_v[i]
            cc = col_v[i]
            rx = plsc.load_gather(px_v, [r])
            ry = plsc.load_gather(py_v, [r])
            rz = plsc.load_gather(pz_v, [r])
            cx = plsc.load_gather(px_v, [cc])
            cy = plsc.load_gather(py_v, [cc])
            cz = plsc.load_gather(pz_v, [cc])
            dx = cx + ox_v[i] - rx
            dy = cy + oy_v[i] - ry
            dz = cz + oz_v[i] - rz
            d2_v[i] = dx * dx + dy * dy + dz * dz
            return 0

        lax.fori_loop(0, GRP, step, 0)
        pltpu.sync_copy(d2_v, d2_hbm.at[wid])

    Np = px.shape[0]
    k = pl.kernel(
        body,
        out_type=jax.ShapeDtypeStruct((NWv, GRP, LANE), jnp.float32),
        mesh=mesh,
        compiler_params=pltpu.CompilerParams(needs_layout_passes=False,
                                             use_tc_tiling_on_sc=False),
        scratch_types=[
            pltpu.VMEM((Np,), jnp.float32),
            pltpu.VMEM((Np,), jnp.float32),
            pltpu.VMEM((Np,), jnp.float32),
            pltpu.VMEM((GRP, LANE), jnp.int32),
            pltpu.VMEM((GRP, LANE), jnp.int32),
            pltpu.VMEM((GRP, LANE), jnp.float32),
            pltpu.VMEM((GRP, LANE), jnp.float32),
            pltpu.VMEM((GRP, LANE), jnp.float32),
            pltpu.VMEM((GRP, LANE), jnp.float32),
        ],
    )
    return k(row3, col3, ox3, oy3, oz3, px, py, pz)


def _filters_call(d2e, d2o, w1t, b1r, w2t, b2r, E, G, GP, H, E_PAD):
    BE = 2048
    nblk = E_PAD // 2 // BE

    def half(d2, w1, b1, w2, b2, base, step, coeff):
        dist = jnp.sqrt(d2)
        offs = lax.broadcasted_iota(jnp.int32, (BE, GP), 1).astype(jnp.float32) * step
        emb = jnp.exp(coeff * (dist[:, None] - offs) ** 2)
        h1 = jnp.dot(emb, w1, preferred_element_type=jnp.float32)
        h1 = _softplus(h1 + b1) - LN2
        Wf = jnp.dot(h1, w2, preferred_element_type=jnp.float32) + b2
        Cc = 0.5 * (jnp.cos(dist * (math.pi / CUTOFF)) + 1.0)
        eidx = base + 2 * lax.broadcasted_iota(jnp.int32, (BE,), 0)
        Cc = jnp.where(eidx < E, Cc, 0.0)
        return Wf * Cc[:, None]

    def body(d2e_ref, d2o_ref, w1t_ref, b1_ref, w2t_ref, b2_ref, wout_ref):
        b = pl.program_id(0)
        step = CUTOFF / (G - 1)
        coeff = -0.5 / (step * step)
        w1 = w1t_ref[...]
        b1 = b1_ref[...]
        w2 = w2t_ref[...]
        b2 = b2_ref[...]
        We = half(d2e_ref[...].reshape(BE), w1, b1, w2, b2, b * 2 * BE, step, coeff)
        Wo = half(d2o_ref[...].reshape(BE), w1, b1, w2, b2, b * 2 * BE + 1, step, coeff)
        wout_ref[...] = _pack_pair(We, Wo)

    return pl.pallas_call(
        body,
        grid=(nblk,),
        in_specs=[
            pl.BlockSpec((BE // 256, 256), lambda b: (b, 0)),
            pl.BlockSpec((BE // 256, 256), lambda b: (b, 0)),
            pl.BlockSpec((GP, H), lambda b: (0, 0)),
            pl.BlockSpec((1, H), lambda b: (0, 0)),
            pl.BlockSpec((H, H), lambda b: (0, 0)),
            pl.BlockSpec((1, H), lambda b: (0, 0)),
        ],
        out_specs=pl.BlockSpec((BE, H), lambda b: (b, 0)),
        out_shape=jax.ShapeDtypeStruct((E_PAD // 2, H), jnp.int32),
    )(d2e, d2o, w1t, b1r, w2t, b2r)


def _message_call(w_e, vh, rowC, colC, N, H, CPT0, CPT1):
    mesh = plsc.VectorSubcoreMesh(core_axis_name="c", subcore_axis_name="s")
    ROWS_PT = N // NS
    ZC = _largest_div(ROWS_PT, CHUNK)

    def body(w_hbm, vh_hbm, row_hbm, col_hbm, out_hbm, accum_sh,
             ri0, ri1, ci0, ci1, wv0, wv1, gv0, gv1, prod_v, cs_v,
             sr0, sr1, sc0, sc1, sw0, sw1, sg0, sg1, sem_s):
        c = lax.axis_index("c")
        s = lax.axis_index("s")
        wid = s * NC + c
        RI = (ri0, ri1)
        CI = (ci0, ci1)
        WV = (wv0, wv1)
        GV = (gv0, gv1)
        SR = (sr0, sr1)
        SC = (sc0, sc1)
        SW = (sw0, sw1)
        SG = (sg0, sg1)
        zz = jnp.zeros((LANE,), jnp.float32)

        @plsc.parallel_loop(0, CHUNK, 1, unroll=4)
        def zrow(i):
            for j in range(H // LANE):
                prod_v[i, pl.ds(j * LANE, LANE)] = zz
        rbase = s * ROWS_PT
        for t in range(ROWS_PT // ZC):
            pltpu.sync_copy(prod_v.at[pl.ds(0, ZC)],
                            accum_sh.at[pl.ds(rbase + t * ZC, ZC)])
        plsc.subcore_barrier()
        cbase = jnp.where(c == 0, s * CPT0, NS * CPT0 + s * CPT1)
        my_cpt = jnp.where(c == 0, CPT0, CPT1)

        def start_idx_w(k, b):
            pltpu.async_copy(row_hbm.at[pl.ds(cbase + k, 1)], RI[b], SR[b])
            pltpu.async_copy(col_hbm.at[pl.ds(cbase + k, 1)], CI[b], SC[b])
            pltpu.async_copy(
                w_hbm.at[pl.ds((cbase + k) * (CHUNK // 2), CHUNK // 2)],
                WV[b], SW[b])

        def wait_r_start_gather(b):
            pltpu.make_async_copy(row_hbm.at[pl.ds(0, 1)], RI[b], SR[b]).wait()
            pltpu.async_copy(vh_hbm.at[RI[b].at[0]], GV[b], SG[b])

        def scatter_wait():
            pltpu.make_async_copy(prod_v, accum_sh.at[cs_v.at[0]], sem_s).wait()

        def process(k, b):
            pltpu.make_async_copy(w_hbm.at[pl.ds(0, CHUNK // 2)], WV[b], SW[b]).wait()
            pltpu.make_async_copy(vh_hbm.at[pl.ds(0, CHUNK)], GV[b], SG[b]).wait()

            @pl.when(k > 0)
            def _():
                scatter_wait()

            @plsc.parallel_loop(0, CHUNK // 2, 1, unroll=4)
            def mrow(r):
                for q in range(H // 32):
                    vA = plsc.bitcast(GV[b][2 * r, pl.ds(LANE * q, LANE)], jnp.bfloat16)
                    vB = plsc.bitcast(GV[b][2 * r + 1, pl.ds(LANE * q, LANE)], jnp.bfloat16)
                    v0e, v1e = plsc.unpack(vA, format=plsc.PackFormat.INTERLEAVED)
                    v0o, v1o = plsc.unpack(vB, format=plsc.PackFormat.INTERLEAVED)
                    wA = plsc.bitcast(WV[b][r, pl.ds(32 * q, LANE)], jnp.bfloat16)
                    wB = plsc.bitcast(WV[b][r, pl.ds(32 * q + LANE, LANE)], jnp.bfloat16)
                    wAe, wAo = plsc.unpack(wA, format=plsc.PackFormat.INTERLEAVED)
                    wBe, wBo = plsc.unpack(wB, format=plsc.PackFormat.INTERLEAVED)
                    prod_v[2 * r, pl.ds(32 * q, LANE)] = wAe * v0e
                    prod_v[2 * r, pl.ds(32 * q + LANE, LANE)] = wBe * v1e
                    prod_v[2 * r + 1, pl.ds(32 * q, LANE)] = wAo * v0o
                    prod_v[2 * r + 1, pl.ds(32 * q + LANE, LANE)] = wBo * v1o
            pltpu.make_async_copy(col_hbm.at[pl.ds(0, 1)], CI[b], SC[b]).wait()
            for t in range(CHUNK // LANE):
                cs_v[0, pl.ds(LANE * t, LANE)] = CI[b][0, pl.ds(LANE * t, LANE)]
            pltpu.async_copy(prod_v, accum_sh.at[cs_v.at[0]], sem_s, add=True)

        start_idx_w(0, 0)
        start_idx_w(1, 1)
        wait_r_start_gather(0)

        def body2(j, _):
            k = 2 * j
            wait_r_start_gather(1)
            process(k, 0)

            @pl.when(k + 2 < my_cpt)
            def _():
                start_idx_w(k + 2, 0)
            process(k + 1, 1)

            @pl.when(k + 3 < my_cpt)
            def _():
                start_idx_w(k + 3, 1)

            @pl.when(k + 2 < my_cpt)
            def _():
                wait_r_start_gather(0)
            return 0

        lax.fori_loop(0, my_cpt // 2, body2, 0)
        scatter_wait()
        plsc.subcore_barrier()
        pltpu.sync_copy(accum_sh.at[pl.ds(rbase, ROWS_PT)],
                        out_hbm.at[c, pl.ds(rbase, ROWS_PT)])

    k = pl.kernel(
        body,
        out_type=jax.ShapeDtypeStruct((NC, N, H), jnp.float32),
        mesh=mesh,
        compiler_params=pltpu.CompilerParams(needs_layout_passes=False,
                                             use_tc_tiling_on_sc=False),
        scratch_types=[
            pltpu.VMEM_SHARED((N, H), jnp.float32),
            pltpu.VMEM((1, CHUNK), jnp.int32),
            pltpu.VMEM((1, CHUNK), jnp.int32),
            pltpu.VMEM((1, CHUNK), jnp.int32),
            pltpu.VMEM((1, CHUNK), jnp.int32),
            pltpu.VMEM((CHUNK // 2, H), jnp.int32),
            pltpu.VMEM((CHUNK // 2, H), jnp.int32),
            pltpu.VMEM((CHUNK, H // 2), jnp.int32),
            pltpu.VMEM((CHUNK, H // 2), jnp.int32),
            pltpu.VMEM((CHUNK, H), jnp.float32),
            pltpu.VMEM((1, CHUNK), jnp.int32),
        ] + [pltpu.SemaphoreType.DMA] * 9,
    )
    return k(w_e, vh, rowC, colC)


def _update_call(part, v, w1t, b1, w2t, b2, lo, hi, N, H):
    BN = _largest_div(N, 1024, mult=8)

    def body(p_ref, v_ref, w1_ref, b1_ref, w2_ref, b2_ref, lo_ref, hi_ref,
             vn_ref, vh_ref):
        out = p_ref[0] + p_ref[1]
        h = _softplus(jnp.dot(out, w1_ref[...], preferred_element_type=jnp.float32)
                      + b1_ref[...]) - LN2
        upd = jnp.dot(h, w2_ref[...], preferred_element_type=jnp.float32) + b2_ref[...]
        vn = v_ref[...] + upd
        vn_ref[...] = vn
        vh_ref[...] = _pack_pair(
            jnp.dot(vn, lo_ref[...], preferred_element_type=jnp.float32),
            jnp.dot(vn, hi_ref[...], preferred_element_type=jnp.float32))

    grid = (N // BN,)
    return pl.pallas_call(
        body,
        grid=grid,
        in_specs=[
            pl.BlockSpec((2, BN, H), lambda b: (0, b, 0)),
            pl.BlockSpec((BN, H), lambda b: (b, 0)),
            pl.BlockSpec((H, H), lambda b: (0, 0)),
            pl.BlockSpec((1, H), lambda b: (0, 0)),
            pl.BlockSpec((H, H), lambda b: (0, 0)),
            pl.BlockSpec((1, H), lambda b: (0, 0)),
            pl.BlockSpec((H, H // 2), lambda b: (0, 0)),
            pl.BlockSpec((H, H // 2), lambda b: (0, 0)),
        ],
        out_specs=[
            pl.BlockSpec((BN, H), lambda b: (b, 0)),
            pl.BlockSpec((BN, H // 2), lambda b: (b, 0)),
        ],
        out_shape=[
            jax.ShapeDtypeStruct((N, H), jnp.float32),
            jax.ShapeDtypeStruct((N, H // 2), jnp.int32),
        ],
    )(part, v, w1t, b1, w2t, b2, lo, hi)


def _vh0_call(v, lo, hi, N, H):
    BN = _largest_div(N, 1024, mult=8)

    def body(v_ref, lo_ref, hi_ref, vh_ref):
        vh_ref[...] = _pack_pair(
            jnp.dot(v_ref[...], lo_ref[...], preferred_element_type=jnp.float32),
            jnp.dot(v_ref[...], hi_ref[...], preferred_element_type=jnp.float32))

    return pl.pallas_call(
        body,
        grid=(N // BN,),
        in_specs=[
            pl.BlockSpec((BN, H), lambda b: (b, 0)),
            pl.BlockSpec((H, H // 2), lambda b: (0, 0)),
            pl.BlockSpec((H, H // 2), lambda b: (0, 0)),
        ],
        out_specs=pl.BlockSpec((BN, H // 2), lambda b: (b, 0)),
        out_shape=jax.ShapeDtypeStruct((N, H // 2), jnp.int32),
    )(v, lo, hi)


def kernel(v, pos, edges, offsets_real, lin_w, mlp_w1, mlp_b1, mlp_w2, mlp_b2,
           v_w1, v_b1, v_w2, v_b2):
    N, H = v.shape
    L, FLT, G = mlp_w1.shape
    E = edges.shape[1]
    assert N % NS == 0 and H % LANE == 0

    sum_pt = -(-E // (NS * CHUNK))
    CPT0 = max(2, (sum_pt // 3) & ~1)
    CPT1 = max(2, sum_pt - CPT0)
    CPT1 += CPT1 % 2
    while NS * (CPT0 + CPT1) * CHUNK < E:
        CPT1 += 2
    NCHT = NS * (CPT0 + CPT1)
    E_PAD = NCHT * CHUNK
    GRP = E_PAD // (NW * LANE)
    GP = -(-G // 8) * 8

    f32 = jnp.float32
    row = jnp.pad(edges[0], (0, E_PAD - E)).astype(jnp.int32)
    col = jnp.pad(edges[1], (0, E_PAD - E)).astype(jnp.int32)
    offp = jnp.pad(offsets_real, ((0, E_PAD - E), (0, 0))).astype(f32)
    row3 = row.reshape(NW, GRP, LANE)
    col3 = col.reshape(NW, GRP, LANE)
    ox3 = offp[:, 0].reshape(NW, GRP, LANE)
    oy3 = offp[:, 1].reshape(NW, GRP, LANE)
    oz3 = offp[:, 2].reshape(NW, GRP, LANE)
    px = pos[:, 0].astype(f32)
    py = pos[:, 1].astype(f32)
    pz = pos[:, 2].astype(f32)

    d2 = _d2_call(row3, col3, ox3, oy3, oz3, px, py, pz)
    d2f = d2.reshape(E_PAD)
    d2e = d2f[0::2].reshape(E_PAD // 512, 256)
    d2o = d2f[1::2].reshape(E_PAD // 512, 256)

    P2 = _interleave_perm(H)
    w1t = jnp.pad(jnp.swapaxes(mlp_w1, 1, 2), ((0, 0), (0, GP - G), (0, 0))).astype(f32)
    b1r = mlp_b1.reshape(L, 1, FLT).astype(f32)
    w2t = jnp.swapaxes(mlp_w2, 1, 2).astype(f32)
    b2r = mlp_b2.reshape(L, 1, FLT).astype(f32)

    rowC = row.reshape(NCHT, CHUNK)
    colC = col.reshape(NCHT, CHUNK)

    v = v.astype(f32)

    def lohi(w):
        lp = jnp.swapaxes(w, 0, 1).astype(f32)[:, P2]
        return lp[:, 0::2], lp[:, 1::2]

    lo0, hi0 = lohi(lin_w[0])
    vh = _vh0_call(v, lo0, hi0, N, H)
    for l in range(L):
        W_l = _filters_call(d2e, d2o, w1t[l], b1r[l], w2t[l], b2r[l],
                            E, G, GP, H, E_PAD)
        part = _message_call(W_l, vh, rowC, colC, N, H, CPT0, CPT1)
        lo, hi = lohi(lin_w[(l + 1) % L])
        v, vh = _update_call(part, v,
                             jnp.swapaxes(v_w1[l], 0, 1).astype(f32),
                             v_b1[l].reshape(1, H).astype(f32),
                             jnp.swapaxes(v_w2[l], 0, 1).astype(f32),
                             v_b2[l].reshape(1, H).astype(f32),
                             lo, hi, N, H)
    return v
